# Initial kernel scaffold; baseline (speedup 1.0000x reference)
import jax
import jax.numpy as jnp
from jax.experimental import pallas as pl


def kernel(input, coeffs, G_rows, G_cols, G_vals, L_rows, L_cols, L_vals, F_rows, F_cols, F_vals, NS, EW):
    raise NotImplementedError("write your pallas kernel here")


# speedup vs baseline: 36.8186x; 36.8186x over previous
import functools

import jax
import jax.numpy as jnp
from jax import lax
from jax.experimental import pallas as pl
from jax.experimental.pallas import tpu as pltpu
from jax.experimental.pallas import tpu_sc as plsc

NV = 40962
NV_PREV = 10242
NF = 81920
C = 256
OUT_CH = 128
BS = 2
LANES = 16
NGRP = C // LANES

NC, NSUB = 2, 16
NW = NC * NSUB

CHUNK = 8
NV_PAD = 41472
NF_PER_W = NF // NW
NV_PER_W = NV_PAD // NW

_MESH = plsc.VectorSubcoreMesh(
    core_axis_name="c", subcore_axis_name="s", num_cores=NC, num_subcores=NSUB)


def _wbcast(ref, i):
  row = ref[pl.ds((i // LANES) * LANES, LANES)]
  return jnp.broadcast_to(row[i % LANES], (LANES,))


def _pad16(n):
  return ((n + LANES - 1) // LANES) * LANES


def _wid():
  return lax.axis_index("s") * NC + lax.axis_index("c")


def _accumulate_row(rows_v, row_base, nnz, w_vecs, out_v, out_row):
  for g in range(NGRP):
    acc = None
    for j in range(nnz):
      r = rows_v[row_base + j, pl.ds(g * LANES, LANES)]
      term = w_vecs[j] * r
      acc = term if acc is None else acc + term
    out_v[out_row, pl.ds(g * LANES, LANES)] = acc


def _accumulate_row2(rows0_v, rows1_v, row_base, nnz, w_vecs, out0_v, out1_v,
                     out_row):
  for g in range(NGRP):
    acc0 = None
    acc1 = None
    for j in range(nnz):
      w = w_vecs[j]
      r0 = rows0_v[row_base + j, pl.ds(g * LANES, LANES)]
      r1 = rows1_v[row_base + j, pl.ds(g * LANES, LANES)]
      t0 = w * r0
      t1 = w * r1
      acc0 = t0 if acc0 is None else acc0 + t0
      acc1 = t1 if acc1 is None else acc1 + t1
    out0_v[out_row, pl.ds(g * LANES, LANES)] = acc0
    out1_v[out_row, pl.ds(g * LANES, LANES)] = acc1


def _build_grad():
  n_idx = CHUNK * 9
  n_chunks = NF_PER_W // CHUNK

  @functools.partial(
      pl.kernel,
      mesh=_MESH,
      out_type=(jax.ShapeDtypeStruct((NF, C), jnp.float32),
                jax.ShapeDtypeStruct((NF, C), jnp.float32)),
      scratch_types=[
          pltpu.VMEM((n_idx,), jnp.int32),
          pltpu.VMEM((_pad16(n_idx),), jnp.float32),
          pltpu.VMEM((_pad16(CHUNK * 3),), jnp.float32),
          pltpu.VMEM((_pad16(CHUNK * 3),), jnp.float32),
          pltpu.VMEM((n_idx, C), jnp.float32),
          pltpu.VMEM((CHUNK, C), jnp.float32),
          pltpu.VMEM((CHUNK, C), jnp.float32),
          pltpu.SemaphoreType.DMA,
      ],
  )
  def grad_k(xt_hbm, gcols_hbm, gvals_hbm, ew_hbm, ns_hbm, oew_hbm, ons_hbm,
             idx_v, gv_v, ew_v, ns_v, rows_v, oew_v, ons_v, sem):
    base = _wid() * NF_PER_W

    def chunk_body(i, carry):
      f0 = base + i * CHUNK
      pltpu.sync_copy(gcols_hbm.at[pl.ds(f0 * 9, n_idx)], idx_v)
      pltpu.sync_copy(gvals_hbm.at[pl.ds(f0 * 9, n_idx)],
                      gv_v.at[pl.ds(0, n_idx)])
      pltpu.sync_copy(ew_hbm.at[pl.ds(f0 * 3, CHUNK * 3)],
                      ew_v.at[pl.ds(0, CHUNK * 3)])
      pltpu.sync_copy(ns_hbm.at[pl.ds(f0 * 3, CHUNK * 3)],
                      ns_v.at[pl.ds(0, CHUNK * 3)])
      pltpu.async_copy(xt_hbm.at[idx_v], rows_v, sem).wait()
      for f in range(CHUNK):
        wew = []
        wns = []
        for j in range(9):
          gv = _wbcast(gv_v, f * 9 + j)
          ew = _wbcast(ew_v, f * 3 + j // 3)
          ns = _wbcast(ns_v, f * 3 + j // 3)
          wew.append(gv * ew)
          wns.append(gv * ns)
        for g in range(NGRP):
          acc_ew = None
          acc_ns = None
          for j in range(9):
            r = rows_v[f * 9 + j, pl.ds(g * LANES, LANES)]
            tew = wew[j] * r
            tns = wns[j] * r
            acc_ew = tew if acc_ew is None else acc_ew + tew
            acc_ns = tns if acc_ns is None else acc_ns + tns
          oew_v[f, pl.ds(g * LANES, LANES)] = acc_ew
          ons_v[f, pl.ds(g * LANES, LANES)] = acc_ns
      pltpu.sync_copy(oew_v, oew_hbm.at[pl.ds(f0, CHUNK)])
      pltpu.sync_copy(ons_v, ons_hbm.at[pl.ds(f0, CHUNK)])
      return carry

    lax.fori_loop(0, n_chunks, chunk_body, 0)

  return grad_k


def _build_combine1(nnz):
  n_idx = CHUNK * nnz
  n_chunks = NV_PER_W // CHUNK

  @functools.partial(
      pl.kernel,
      mesh=_MESH,
      out_type=jax.ShapeDtypeStruct((NV_PAD, C), jnp.float32),
      scratch_types=[
          pltpu.VMEM((n_idx,), jnp.int32),
          pltpu.VMEM((_pad16(n_idx),), jnp.float32),
          pltpu.VMEM((n_idx, C), jnp.float32),
          pltpu.VMEM((CHUNK, C), jnp.float32),
          pltpu.SemaphoreType.DMA,
      ],
  )
  def comb_k(tab_hbm, cols_hbm, vals_hbm, out_hbm, idx_v, w_v, rows_v, out_v,
             sem):
    base = _wid() * NV_PER_W

    def chunk_body(i, carry):
      r0 = base + i * CHUNK
      pltpu.sync_copy(cols_hbm.at[pl.ds(r0 * nnz, n_idx)], idx_v)
      pltpu.sync_copy(vals_hbm.at[pl.ds(r0 * nnz, n_idx)],
                      w_v.at[pl.ds(0, n_idx)])
      pltpu.async_copy(tab_hbm.at[idx_v], rows_v, sem).wait()
      for r in range(CHUNK):
        wv = [_wbcast(w_v, r * nnz + j) for j in range(nnz)]
        _accumulate_row(rows_v, r * nnz, nnz, wv, out_v, r)
      pltpu.sync_copy(out_v, out_hbm.at[pl.ds(r0, CHUNK)])
      return carry

    lax.fori_loop(0, n_chunks, chunk_body, 0)

  return comb_k


def _build_combine2(nnz):
  n_idx = CHUNK * nnz
  n_chunks = NV_PER_W // CHUNK

  @functools.partial(
      pl.kernel,
      mesh=_MESH,
      out_type=(jax.ShapeDtypeStruct((NV_PAD, C), jnp.float32),
                jax.ShapeDtypeStruct((NV_PAD, C), jnp.float32)),
      scratch_types=[
          pltpu.VMEM((n_idx,), jnp.int32),
          pltpu.VMEM((_pad16(n_idx),), jnp.float32),
          pltpu.VMEM((n_idx, C), jnp.float32),
          pltpu.VMEM((n_idx, C), jnp.float32),
          pltpu.VMEM((CHUNK, C), jnp.float32),
          pltpu.VMEM((CHUNK, C), jnp.float32),
          pltpu.SemaphoreType.DMA,
          pltpu.SemaphoreType.DMA,
      ],
  )
  def comb2_k(tab0_hbm, tab1_hbm, cols_hbm, vals_hbm, out0_hbm, out1_hbm,
              idx_v, w_v, rows0_v, rows1_v, out0_v, out1_v, sem0, sem1):
    base = _wid() * NV_PER_W

    def chunk_body(i, carry):
      r0 = base + i * CHUNK
      pltpu.sync_copy(cols_hbm.at[pl.ds(r0 * nnz, n_idx)], idx_v)
      pltpu.sync_copy(vals_hbm.at[pl.ds(r0 * nnz, n_idx)],
                      w_v.at[pl.ds(0, n_idx)])
      cp0 = pltpu.async_copy(tab0_hbm.at[idx_v], rows0_v, sem0)
      cp1 = pltpu.async_copy(tab1_hbm.at[idx_v], rows1_v, sem1)
      cp0.wait()
      cp1.wait()
      for r in range(CHUNK):
        wv = [_wbcast(w_v, r * nnz + j) for j in range(nnz)]
        _accumulate_row2(rows0_v, rows1_v, r * nnz, nnz, wv, out0_v, out1_v, r)
      pltpu.sync_copy(out0_v, out0_hbm.at[pl.ds(r0, CHUNK)])
      pltpu.sync_copy(out1_v, out1_hbm.at[pl.ds(r0, CHUNK)])
      return carry

    lax.fori_loop(0, n_chunks, chunk_body, 0)

  return comb2_k


_GRAD_K = _build_grad()
_LAP_K = _build_combine1(7)
_F2V_K = _build_combine2(6)

_NB = 512
_NBLK = NV_PAD // _NB


def _tc_body(x_ref, l_ref, e_ref, n_ref, c_ref, o_ref):
  feats = (x_ref, l_ref, e_ref, n_ref)
  acc = None
  for j in range(4):
    t = lax.dot_general(c_ref[j], feats[j][...], (((0,), (1,)), ((), ())),
                        preferred_element_type=jnp.float32)
    acc = t if acc is None else acc + t
  o_ref[...] = acc[None]


def _tc_matmul(x_t, lap, gv_ew, gv_ns, cj):
  feat_spec = pl.BlockSpec((_NB, 128), lambda b, i: (i, b))
  return pl.pallas_call(
      _tc_body,
      grid=(BS, _NBLK),
      in_specs=[feat_spec, feat_spec, feat_spec, feat_spec,
                pl.BlockSpec((4, 128, OUT_CH), lambda b, i: (0, 0, 0))],
      out_specs=pl.BlockSpec((1, OUT_CH, _NB), lambda b, i: (b, 0, i)),
      out_shape=jax.ShapeDtypeStruct((BS, OUT_CH, NV_PAD), jnp.float32),
  )(x_t, lap, gv_ew, gv_ns, cj)


def kernel(input, coeffs, G_rows, G_cols, G_vals, L_rows, L_cols, L_vals,
           F_rows, F_cols, F_vals, NS, EW):
  bs, ch, _ = input.shape
  x_t = jnp.concatenate([
      input.reshape(bs * ch, NV_PREV).T,
      jnp.ones((NV - NV_PREV, C), jnp.float32),
      jnp.zeros((NV_PAD - NV, C), jnp.float32),
  ], axis=0)

  gcols = G_cols.reshape(3, NF, 3).transpose(1, 0, 2).reshape(-1)
  gvals = G_vals.reshape(3, NF, 3).transpose(1, 0, 2).reshape(-1)
  gf_ew, gf_ns = _GRAD_K(x_t, gcols, gvals, EW.reshape(-1), NS.reshape(-1))

  pad_rows = NV_PAD - NV
  lcols = jnp.pad(L_cols.reshape(NV, 7), ((0, pad_rows), (0, 0))).reshape(-1)
  lvals = jnp.pad(L_vals.reshape(NV, 7), ((0, pad_rows), (0, 0))).reshape(-1)
  lap = _LAP_K(x_t, lcols, lvals)

  fcols = jnp.pad(F_cols.reshape(NV, 6), ((0, pad_rows), (0, 0))).reshape(-1)
  fvals = jnp.pad(F_vals.reshape(NV, 6), ((0, pad_rows), (0, 0))).reshape(-1)
  gv_ew, gv_ns = _F2V_K(gf_ew, gf_ns, fcols, fvals)

  cj = coeffs.reshape(ch, 4, OUT_CH).transpose(1, 0, 2)
  out = _tc_matmul(x_t, lap, gv_ew, gv_ns, cj)
  return out[:, :, :NV]


# speedup vs baseline: 65.7341x; 1.7853x over previous
import functools

import jax
import jax.numpy as jnp
from jax import lax
from jax.experimental import pallas as pl
from jax.experimental.pallas import tpu as pltpu
from jax.experimental.pallas import tpu_sc as plsc

NV = 40962
NV_PREV = 10242
NF = 81920
C = 256
OUT_CH = 128
BS = 2
LANES = 16
NGRP = C // LANES
WCOLS = 16

NC, NSUB = 2, 16
NW = NC * NSUB

ROWS_PER_ITER = 16
NV_PAD = 41472
NF_PER_W = NF // NW
NV_PER_W = NV_PAD // NW

_MESH = plsc.VectorSubcoreMesh(
    core_axis_name="c", subcore_axis_name="s", num_cores=NC, num_subcores=NSUB)


def _wid():
  return lax.axis_index("s") * NC + lax.axis_index("c")


def _bcast(x):
  return jnp.broadcast_to(x, (LANES,))


def _row_combine(wrow, weight_fns, row_bufs, row_base, nnz, out_bufs, out_row):
  wvecs = [[_bcast(weight_fns[t](j)) for j in range(nnz)]
           for t in range(len(out_bufs))]
  for g in range(NGRP):
    accs = [None] * len(out_bufs)
    rb_cache = {}
    for j in range(nnz):
      for t in range(len(out_bufs)):
        src = row_bufs[t if len(row_bufs) > 1 else 0]
        key = (id(src), j)
        if key not in rb_cache:
          rb_cache[key] = src[row_base + j, pl.ds(g * LANES, LANES)]
        term = wvecs[t][j] * rb_cache[key]
        accs[t] = term if accs[t] is None else accs[t] + term
    for t in range(len(out_bufs)):
      out_bufs[t][out_row, pl.ds(g * LANES, LANES)] = accs[t]


def _build_sc_kernel(*, n_rows_out, rows_per_w, nnz, supc, n_tables, n_outs,
                     weight_fns_builder, table_shapes):
  n_iters = rows_per_w // ROWS_PER_ITER
  n_sup = n_iters // supc
  assert n_sup * supc == n_iters
  idx_per_iter = ROWS_PER_ITER * nnz
  n_gath = -(-idx_per_iter // 128)
  assert idx_per_iter % n_gath == 0
  idx_per_gath = idx_per_iter // n_gath
  assert idx_per_gath % 8 == 0 and idx_per_iter % 8 == 0

  out_types = tuple(jax.ShapeDtypeStruct((n_rows_out, C), jnp.float32)
                    for _ in range(n_outs))
  scratch = (
      [pltpu.VMEM((supc * idx_per_iter,), jnp.int32)] +
      [pltpu.VMEM((supc * ROWS_PER_ITER * WCOLS,), jnp.float32)] +
      [pltpu.VMEM((idx_per_iter, C), jnp.float32)
       for _ in range(2 * n_tables)] +
      [pltpu.VMEM((ROWS_PER_ITER, C), jnp.float32)
       for _ in range(2 * n_outs)] +
      [pltpu.SemaphoreType.DMA for _ in range(2)] +
      [pltpu.SemaphoreType.DMA for _ in range(2 * n_outs)]
  )

  @functools.partial(pl.kernel, mesh=_MESH,
                     out_type=out_types if n_outs > 1 else out_types[0],
                     scratch_types=scratch)
  def sc_k(*refs):
    tabs = refs[:n_tables]
    cols_hbm, wpk_hbm = refs[n_tables:n_tables + 2]
    outs_hbm = refs[n_tables + 2:n_tables + 2 + n_outs]
    pos = n_tables + 2 + n_outs
    colbuf, wbuf = refs[pos], refs[pos + 1]
    pos += 2
    rowbufs = [refs[pos:pos + n_tables], refs[pos + n_tables:pos + 2 * n_tables]]
    pos += 2 * n_tables
    outbufs = [refs[pos:pos + n_outs], refs[pos + n_outs:pos + 2 * n_outs]]
    pos += 2 * n_outs
    gsems = refs[pos:pos + 2]
    wsems = [refs[pos + 2:pos + 2 + n_outs], refs[pos + 2 + n_outs:pos + 2 + 2 * n_outs]]

    base = _wid() * rows_per_w

    def issue_gathers(i, p):
      hs = []
      for t in range(n_tables):
        for gidx in range(n_gath):
          lo = i * idx_per_iter + gidx * idx_per_gath
          hs.append(pltpu.async_copy(
              tabs[t].at[colbuf.at[pl.ds(lo, idx_per_gath)]],
              rowbufs[p][t].at[pl.ds(gidx * idx_per_gath, idx_per_gath)]
              if n_tables == 1 else rowbufs[p][t],
              gsems[p]))
      return hs

    def sup_body(s, carry):
      r0 = base + s * (supc * ROWS_PER_ITER)
      pltpu.sync_copy(cols_hbm.at[pl.ds(r0 * nnz, supc * idx_per_iter)],
                      colbuf)
      pltpu.sync_copy(wpk_hbm.at[pl.ds(r0 * WCOLS, supc * ROWS_PER_ITER * WCOLS)],
                      wbuf)
      gh = {0: issue_gathers(0, 0)}
      wh = {}
      for i in range(supc):
        p = i % 2
        if i + 1 < supc:
          gh[i + 1] = issue_gathers(i + 1, (i + 1) % 2)
        for h in gh.pop(i):
          h.wait()
        if i - 2 in wh:
          for h in wh.pop(i - 2):
            h.wait()

        def row_body(r, c2):
          wrow = wbuf[pl.ds(i * ROWS_PER_ITER * WCOLS + r * WCOLS, WCOLS)]
          _row_combine(wrow, weight_fns_builder(wrow), rowbufs[p], r * nnz,
                       nnz, outbufs[p], r)
          return c2

        lax.fori_loop(0, ROWS_PER_ITER, row_body, 0)
        ri = r0 + i * ROWS_PER_ITER
        wh[i] = [pltpu.async_copy(outbufs[p][t],
                                  outs_hbm[t].at[pl.ds(ri, ROWS_PER_ITER)],
                                  wsems[p][t])
                 for t in range(n_outs)]
      for k in sorted(wh):
        for h in wh.pop(k):
          h.wait()
      return carry

    lax.fori_loop(0, n_sup, sup_body, 0)

  return sc_k


def _grad_weights(wrow):
  return [lambda j: wrow[j] * wrow[9 + j // 3],
          lambda j: wrow[j] * wrow[12 + j // 3]]


def _plain_weights(wrow):
  return [lambda j: wrow[j], lambda j: wrow[j]]


_GRAD_K = _build_sc_kernel(
    n_rows_out=NF, rows_per_w=NF_PER_W, nnz=9, supc=8, n_tables=1, n_outs=2,
    weight_fns_builder=_grad_weights, table_shapes=None)
_LAP_K = _build_sc_kernel(
    n_rows_out=NV_PAD, rows_per_w=NV_PER_W, nnz=7, supc=9, n_tables=1,
    n_outs=1, weight_fns_builder=_plain_weights, table_shapes=None)
_F2V_K = _build_sc_kernel(
    n_rows_out=NV_PAD, rows_per_w=NV_PER_W, nnz=6, supc=9, n_tables=2,
    n_outs=2, weight_fns_builder=_plain_weights, table_shapes=None)

_NB = 512
_NBLK = NV_PAD // _NB


def _tc_body(x_ref, l_ref, e_ref, n_ref, c_ref, o_ref):
  feats = (x_ref, l_ref, e_ref, n_ref)
  acc = None
  for j in range(4):
    t = lax.dot_general(c_ref[j], feats[j][...], (((0,), (1,)), ((), ())),
                        preferred_element_type=jnp.float32)
    acc = t if acc is None else acc + t
  o_ref[...] = acc[None]


def _tc_matmul(x_t, lap, gv_ew, gv_ns, cj):
  feat_spec = pl.BlockSpec((_NB, 128), lambda b, i: (i, b))
  return pl.pallas_call(
      _tc_body,
      grid=(BS, _NBLK),
      in_specs=[feat_spec, feat_spec, feat_spec, feat_spec,
                pl.BlockSpec((4, 128, OUT_CH), lambda b, i: (0, 0, 0))],
      out_specs=pl.BlockSpec((1, OUT_CH, _NB), lambda b, i: (b, 0, i)),
      out_shape=jax.ShapeDtypeStruct((BS, OUT_CH, NV), jnp.float32),
  )(x_t, lap, gv_ew, gv_ns, cj)


def kernel(input, coeffs, G_rows, G_cols, G_vals, L_rows, L_cols, L_vals,
           F_rows, F_cols, F_vals, NS, EW):
  bs, ch, _ = input.shape
  x_t = jnp.concatenate([
      input.reshape(bs * ch, NV_PREV).T,
      jnp.ones((NV - NV_PREV, C), jnp.float32),
      jnp.zeros((NV_PAD - NV, C), jnp.float32),
  ], axis=0)

  gcols = G_cols.reshape(3, NF, 3).transpose(1, 0, 2).reshape(-1)
  gvals = G_vals.reshape(3, NF, 3).transpose(1, 0, 2).reshape(NF, 9)
  gw = jnp.concatenate([gvals, EW, NS, jnp.zeros((NF, 1), jnp.float32)],
                       axis=1).reshape(-1)
  gf_ew, gf_ns = _GRAD_K(x_t, gcols, gw)

  pad_rows = NV_PAD - NV
  lcols = jnp.pad(L_cols.reshape(NV, 7), ((0, pad_rows), (0, 0))).reshape(-1)
  lw = jnp.pad(L_vals.reshape(NV, 7),
               ((0, pad_rows), (0, WCOLS - 7))).reshape(-1)
  lap = _LAP_K(x_t, lcols, lw)

  fcols = jnp.pad(F_cols.reshape(NV, 6), ((0, pad_rows), (0, 0))).reshape(-1)
  fw = jnp.pad(F_vals.reshape(NV, 6),
               ((0, pad_rows), (0, WCOLS - 6))).reshape(-1)
  gv_ew, gv_ns = _F2V_K(gf_ew, gf_ns, fcols, fw)

  cj = coeffs.reshape(ch, 4, OUT_CH).transpose(1, 0, 2)
  return _tc_matmul(x_t, lap, gv_ew, gv_ns, cj)


# speedup vs baseline: 82.6598x; 1.2575x over previous
import functools

import jax
import jax.numpy as jnp
from jax import lax
from jax.experimental import pallas as pl
from jax.experimental.pallas import tpu as pltpu
from jax.experimental.pallas import tpu_sc as plsc

NV = 40962
NV_PREV = 10242
NF = 81920
C = 256
OUT_CH = 128
BS = 2
LANES = 16
NGRP = C // LANES

NC, NSUB = 2, 16
NW = NC * NSUB

RPI = 16
NV_PAD = 41472
NF_PER_W = NF // NW
NV_PER_W = NV_PAD // NW

_MESH = plsc.VectorSubcoreMesh(
    core_axis_name="c", subcore_axis_name="s", num_cores=NC, num_subcores=NSUB)


def _wid():
  return lax.axis_index("s") * NC + lax.axis_index("c")


def _bcast(x):
  return jnp.broadcast_to(x, (LANES,))


def _build_grad(supc=8):
  n_iters = NF_PER_W // RPI
  n_sup = n_iters // supc
  sec = supc * RPI * 3

  @functools.partial(
      pl.kernel, mesh=_MESH,
      out_type=(jax.ShapeDtypeStruct((NF, C), jnp.float32),
                jax.ShapeDtypeStruct((NF, C), jnp.float32)),
      scratch_types=(
          [pltpu.VMEM((3 * sec,), jnp.int32)] +
          [pltpu.VMEM((5 * sec + LANES,), jnp.float32)] +
          [pltpu.VMEM((RPI * 9, C), jnp.float32) for _ in range(2)] +
          [pltpu.VMEM((RPI, C), jnp.float32) for _ in range(4)] +
          [pltpu.SemaphoreType.DMA for _ in range(6)]),
  )
  def grad_k(xt_hbm, gcols_hbm, gvals_hbm, ew_hbm, ns_hbm, oew_hbm, ons_hbm,
             colbuf, wbuf, rows0, rows1, oew0, oew1, ons0, ons1,
             gsem0, gsem1, wsem_ew0, wsem_ew1, wsem_ns0, wsem_ns1):
    rowsb = (rows0, rows1)
    oewb = (oew0, oew1)
    onsb = (ons0, ons1)
    gsems = (gsem0, gsem1)
    wsems = ((wsem_ew0, wsem_ns0), (wsem_ew1, wsem_ns1))
    base = _wid() * NF_PER_W

    def issue_gathers(i, p):
      return [pltpu.async_copy(
          xt_hbm.at[colbuf.at[pl.ds(d * sec + i * 48, 48)]],
          rowsb[p].at[pl.ds(d * 48, 48)], gsems[p]) for d in range(3)]

    def sup_body(s, carry):
      f0 = base + s * (supc * RPI)
      for d in range(3):
        pltpu.sync_copy(gcols_hbm.at[pl.ds(d * 3 * NF + f0 * 3, sec)],
                        colbuf.at[pl.ds(d * sec, sec)])
        pltpu.sync_copy(gvals_hbm.at[pl.ds(d * 3 * NF + f0 * 3, sec)],
                        wbuf.at[pl.ds(d * sec, sec)])
      pltpu.sync_copy(ew_hbm.at[pl.ds(f0 * 3, sec)],
                      wbuf.at[pl.ds(3 * sec, sec)])
      pltpu.sync_copy(ns_hbm.at[pl.ds(f0 * 3, sec)],
                      wbuf.at[pl.ds(4 * sec, sec)])
      gh = {0: issue_gathers(0, 0)}
      wh = {}
      for i in range(supc):
        p = i % 2
        if i + 1 < supc:
          gh[i + 1] = issue_gathers(i + 1, (i + 1) % 2)
        for h in gh.pop(i):
          h.wait()
        if i - 2 in wh:
          for h in wh.pop(i - 2):
            h.wait()

        def row_body(r, c2):
          off = i * (RPI * 3) + r * 3
          gvv = [wbuf[pl.ds(d * sec + off, LANES)] for d in range(3)]
          eww = wbuf[pl.ds(3 * sec + off, LANES)]
          nsw = wbuf[pl.ds(4 * sec + off, LANES)]
          wew = [_bcast(gvv[d][k] * eww[d]) for d in range(3) for k in range(3)]
          wns = [_bcast(gvv[d][k] * nsw[d]) for d in range(3) for k in range(3)]
          for g in range(NGRP):
            acc_ew = None
            acc_ns = None
            for d in range(3):
              for k in range(3):
                rv = rowsb[p][d * 48 + r * 3 + k, pl.ds(g * LANES, LANES)]
                tew = wew[d * 3 + k] * rv
                tns = wns[d * 3 + k] * rv
                acc_ew = tew if acc_ew is None else acc_ew + tew
                acc_ns = tns if acc_ns is None else acc_ns + tns
            oewb[p][r, pl.ds(g * LANES, LANES)] = acc_ew
            onsb[p][r, pl.ds(g * LANES, LANES)] = acc_ns
          return c2

        lax.fori_loop(0, RPI, row_body, 0)
        ri = f0 + i * RPI
        wh[i] = [
            pltpu.async_copy(oewb[p], oew_hbm.at[pl.ds(ri, RPI)], wsems[p][0]),
            pltpu.async_copy(onsb[p], ons_hbm.at[pl.ds(ri, RPI)], wsems[p][1]),
        ]
      for kk in sorted(wh):
        for h in wh.pop(kk):
          h.wait()
      return carry

    lax.fori_loop(0, n_sup, sup_body, 0)

  return grad_k


def _build_combine(nnz, n_tables, n_outs, supc=9):
  n_iters = NV_PER_W // RPI
  n_sup = n_iters // supc
  ipi = RPI * nnz
  assert ipi <= 128 and ipi % 8 == 0

  scratch = (
      [pltpu.VMEM((supc * ipi,), jnp.int32)] +
      [pltpu.VMEM((supc * ipi + LANES,), jnp.float32)] +
      [pltpu.VMEM((ipi, C), jnp.float32) for _ in range(2 * n_tables)] +
      [pltpu.VMEM((RPI, C), jnp.float32) for _ in range(2 * n_outs)] +
      [pltpu.SemaphoreType.DMA for _ in range(2 + 2 * n_outs)])
  out_types = tuple(jax.ShapeDtypeStruct((NV_PAD, C), jnp.float32)
                    for _ in range(n_outs))

  @functools.partial(pl.kernel, mesh=_MESH,
                     out_type=out_types if n_outs > 1 else out_types[0],
                     scratch_types=scratch)
  def comb_k(*refs):
    tabs = refs[:n_tables]
    cols_hbm, vals_hbm = refs[n_tables:n_tables + 2]
    outs_hbm = refs[n_tables + 2:n_tables + 2 + n_outs]
    pos = n_tables + 2 + n_outs
    colbuf, wbuf = refs[pos], refs[pos + 1]
    pos += 2
    rowsb = (refs[pos:pos + n_tables], refs[pos + n_tables:pos + 2 * n_tables])
    pos += 2 * n_tables
    outb = (refs[pos:pos + n_outs], refs[pos + n_outs:pos + 2 * n_outs])
    pos += 2 * n_outs
    gsems = refs[pos:pos + 2]
    wsems = (refs[pos + 2:pos + 2 + n_outs],
             refs[pos + 2 + n_outs:pos + 2 + 2 * n_outs])
    base = _wid() * NV_PER_W

    def issue_gathers(i, p):
      return [pltpu.async_copy(
          tabs[t].at[colbuf.at[pl.ds(i * ipi, ipi)]], rowsb[p][t], gsems[p])
          for t in range(n_tables)]

    def sup_body(s, carry):
      r0 = base + s * (supc * RPI)
      pltpu.sync_copy(cols_hbm.at[pl.ds(r0 * nnz, supc * ipi)], colbuf)
      pltpu.sync_copy(vals_hbm.at[pl.ds(r0 * nnz, supc * ipi)],
                      wbuf.at[pl.ds(0, supc * ipi)])
      gh = {0: issue_gathers(0, 0)}
      wh = {}
      for i in range(supc):
        p = i % 2
        if i + 1 < supc:
          gh[i + 1] = issue_gathers(i + 1, (i + 1) % 2)
        for h in gh.pop(i):
          h.wait()
        if i - 2 in wh:
          for h in wh.pop(i - 2):
            h.wait()

        def row_body(r, c2):
          wrow = wbuf[pl.ds(i * ipi + r * nnz, LANES)]
          wv = [_bcast(wrow[j]) for j in range(nnz)]
          for g in range(NGRP):
            accs = [None] * n_outs
            for j in range(nnz):
              for t in range(n_outs):
                rv = rowsb[p][min(t, n_tables - 1)][r * nnz + j,
                                                    pl.ds(g * LANES, LANES)]
                term = wv[j] * rv
                accs[t] = term if accs[t] is None else accs[t] + term
            for t in range(n_outs):
              outb[p][t][r, pl.ds(g * LANES, LANES)] = accs[t]
          return c2

        lax.fori_loop(0, RPI, row_body, 0)
        ri = r0 + i * RPI
        wh[i] = [pltpu.async_copy(outb[p][t], outs_hbm[t].at[pl.ds(ri, RPI)],
                                  wsems[p][t]) for t in range(n_outs)]
      for kk in sorted(wh):
        for h in wh.pop(kk):
          h.wait()
      return carry

    lax.fori_loop(0, n_sup, sup_body, 0)

  return comb_k


_GRAD_K = _build_grad()
_LAP_K = _build_combine(7, n_tables=1, n_outs=1)
_F2V_K = _build_combine(6, n_tables=2, n_outs=2)

_NB = 512
_NBLK = NV_PAD // _NB


def _xt_body(in_ref, o_ref):
  i = pl.program_id(0)
  t = in_ref[...].T
  rowv = lax.broadcasted_iota(jnp.int32, (_NB, C), 0) + i * _NB
  o_ref[...] = jnp.where(rowv < NV_PREV, t,
                         jnp.where(rowv < NV, 1.0, 0.0))


def _build_xt(input2d):
  n_in_blk = -(-NV_PREV // _NB) - 1
  return pl.pallas_call(
      _xt_body,
      grid=(_NBLK,),
      in_specs=[pl.BlockSpec((C, _NB),
                             lambda i: (0, jnp.minimum(i, n_in_blk)))],
      out_specs=pl.BlockSpec((_NB, C), lambda i: (i, 0)),
      out_shape=jax.ShapeDtypeStruct((NV_PAD, C), jnp.float32),
  )(input2d)


def _tc_body(x_ref, l_ref, e_ref, n_ref, c_ref, o_ref):
  feats = (x_ref, l_ref, e_ref, n_ref)
  acc = None
  for j in range(4):
    t = lax.dot_general(c_ref[j], feats[j][...], (((0,), (1,)), ((), ())),
                        preferred_element_type=jnp.float32)
    acc = t if acc is None else acc + t
  o_ref[...] = acc[None]


def _tc_matmul(x_t, lap, gv_ew, gv_ns, cj):
  feat_spec = pl.BlockSpec((_NB, 128), lambda b, i: (i, b))
  return pl.pallas_call(
      _tc_body,
      grid=(BS, _NBLK),
      in_specs=[feat_spec, feat_spec, feat_spec, feat_spec,
                pl.BlockSpec((4, 128, OUT_CH), lambda b, i: (0, 0, 0))],
      out_specs=pl.BlockSpec((1, OUT_CH, _NB), lambda b, i: (b, 0, i)),
      out_shape=jax.ShapeDtypeStruct((BS, OUT_CH, NV), jnp.float32),
  )(x_t, lap, gv_ew, gv_ns, cj)


def kernel(input, coeffs, G_rows, G_cols, G_vals, L_rows, L_cols, L_vals,
           F_rows, F_cols, F_vals, NS, EW):
  bs, ch, _ = input.shape
  x_t = _build_xt(input.reshape(bs * ch, NV_PREV))

  gf_ew, gf_ns = _GRAD_K(x_t, G_cols, G_vals, EW.reshape(-1), NS.reshape(-1))

  padl = (NV_PAD - NV) * 7
  lap = _LAP_K(x_t, jnp.pad(L_cols, (0, padl)), jnp.pad(L_vals, (0, padl)))

  padf = (NV_PAD - NV) * 6
  gv_ew, gv_ns = _F2V_K(gf_ew, gf_ns, jnp.pad(F_cols, (0, padf)),
                        jnp.pad(F_vals, (0, padf)))

  cj = coeffs.reshape(ch, 4, OUT_CH).transpose(1, 0, 2)
  return _tc_matmul(x_t, lap, gv_ew, gv_ns, cj)


# speedup vs baseline: 85.7531x; 1.0374x over previous
import functools

import jax
import jax.numpy as jnp
from jax import lax
from jax.experimental import pallas as pl
from jax.experimental.pallas import tpu as pltpu
from jax.experimental.pallas import tpu_sc as plsc

NV = 40962
NV_PREV = 10242
NF = 81920
C = 256
OUT_CH = 128
BS = 2
LANES = 16
NGRP = C // LANES

NC, NSUB = 2, 16
NW = NC * NSUB

RPI = 16
NV_PAD = 41472
NF_PER_W = NF // NW
NV_PER_W = NV_PAD // NW

GSEC = 3 * NF
LSEC = NV_PAD * 7
FSEC = NV_PAD * 6
OFF_L, OFF_F = 0, LSEC

_MESH = plsc.VectorSubcoreMesh(
    core_axis_name="c", subcore_axis_name="s", num_cores=NC, num_subcores=NSUB)


def _wid():
  return lax.axis_index("s") * NC + lax.axis_index("c")


def _bcast(x):
  return jnp.broadcast_to(x, (LANES,))


def _wvec(ref, off):
  return ref[pl.ds(off, LANES)]


def _build_grad(supc=8):
  n_iters = NF_PER_W // RPI
  n_sup = n_iters // supc
  sec = supc * RPI * 3

  @functools.partial(
      pl.kernel, mesh=_MESH,
      out_type=(jax.ShapeDtypeStruct((NF, C), jnp.float32),
                jax.ShapeDtypeStruct((NF, C), jnp.float32)),
      scratch_types=(
          [pltpu.VMEM((3 * sec,), jnp.int32)] +
          [pltpu.VMEM((5 * sec + LANES,), jnp.float32)] +
          [pltpu.VMEM((RPI * 9, C), jnp.float32) for _ in range(2)] +
          [pltpu.VMEM((RPI, C), jnp.float32) for _ in range(4)] +
          [pltpu.SemaphoreType.DMA for _ in range(7)]),
  )
  def grad_k(xt_hbm, gcols_hbm, wmeta_hbm, oew_hbm, ons_hbm,
             colbuf, wbuf, rows0, rows1, oew0, oew1, ons0, ons1,
             gsem0, gsem1, wsem_ew0, wsem_ew1, wsem_ns0, wsem_ns1, msem):
    rowsb = (rows0, rows1)
    oewb = (oew0, oew1)
    onsb = (ons0, ons1)
    gsems = (gsem0, gsem1)
    wsems = ((wsem_ew0, wsem_ns0), (wsem_ew1, wsem_ns1))
    base = _wid() * NF_PER_W

    def issue_gathers(i, p):
      return [pltpu.async_copy(
          xt_hbm.at[colbuf.at[pl.ds(d * sec + i * 48, 48)]],
          rowsb[p].at[pl.ds(d * 48, 48)], gsems[p]) for d in range(3)]

    def sup_body(s, carry):
      f0 = base + s * (supc * RPI)
      mh = [pltpu.async_copy(gcols_hbm.at[pl.ds(d * GSEC + f0 * 3, sec)],
                             colbuf.at[pl.ds(d * sec, sec)], msem)
            for d in range(3)]
      mh += [pltpu.async_copy(wmeta_hbm.at[pl.ds(k * GSEC + f0 * 3, sec)],
                              wbuf.at[pl.ds(k * sec, sec)], msem)
             for k in range(5)]
      for h in mh:
        h.wait()
      gh = {0: issue_gathers(0, 0)}
      wh = {}
      for i in range(supc):
        p = i % 2
        if i + 1 < supc:
          gh[i + 1] = issue_gathers(i + 1, (i + 1) % 2)
        for h in gh.pop(i):
          h.wait()
        if i - 2 in wh:
          for h in wh.pop(i - 2):
            h.wait()

        def row_body(r, c2):
          off = i * (RPI * 3) + r * 3
          gvv = [_wvec(wbuf, d * sec + off) for d in range(3)]
          eww = _wvec(wbuf, 3 * sec + off)
          nsw = _wvec(wbuf, 4 * sec + off)
          wew = [_bcast(gvv[d][k] * eww[d]) for d in range(3) for k in range(3)]
          wns = [_bcast(gvv[d][k] * nsw[d]) for d in range(3) for k in range(3)]
          for g in range(NGRP):
            acc_ew = None
            acc_ns = None
            for d in range(3):
              for k in range(3):
                rv = rowsb[p][d * 48 + r * 3 + k, pl.ds(g * LANES, LANES)]
                tew = wew[d * 3 + k] * rv
                tns = wns[d * 3 + k] * rv
                acc_ew = tew if acc_ew is None else acc_ew + tew
                acc_ns = tns if acc_ns is None else acc_ns + tns
            oewb[p][r, pl.ds(g * LANES, LANES)] = acc_ew
            onsb[p][r, pl.ds(g * LANES, LANES)] = acc_ns
          return c2

        lax.fori_loop(0, RPI, row_body, 0)
        ri = f0 + i * RPI
        wh[i] = [
            pltpu.async_copy(oewb[p], oew_hbm.at[pl.ds(ri, RPI)], wsems[p][0]),
            pltpu.async_copy(onsb[p], ons_hbm.at[pl.ds(ri, RPI)], wsems[p][1]),
        ]
      for kk in sorted(wh):
        for h in wh.pop(kk):
          h.wait()
      return carry

    lax.fori_loop(0, n_sup, sup_body, 0)

  return grad_k


def _build_combine(nnz, off, n_tables, n_outs, supc=9):
  n_iters = NV_PER_W // RPI
  n_sup = n_iters // supc
  ipi = RPI * nnz
  mlen = supc * ipi
  assert ipi <= 128 and ipi % 8 == 0

  scratch = (
      [pltpu.VMEM((mlen,), jnp.int32)] +
      [pltpu.VMEM((mlen + LANES,), jnp.float32)] +
      [pltpu.VMEM((ipi, C), jnp.float32) for _ in range(2 * n_tables)] +
      [pltpu.VMEM((RPI, C), jnp.float32) for _ in range(2 * n_outs)] +
      [pltpu.SemaphoreType.DMA for _ in range(3 + 2 * n_outs)])
  out_types = tuple(jax.ShapeDtypeStruct((NV_PAD, C), jnp.float32)
                    for _ in range(n_outs))

  @functools.partial(pl.kernel, mesh=_MESH,
                     out_type=out_types if n_outs > 1 else out_types[0],
                     scratch_types=scratch)
  def comb_k(*refs):
    tabs = refs[:n_tables]
    mcols_hbm, mvals_hbm = refs[n_tables:n_tables + 2]
    outs_hbm = refs[n_tables + 2:n_tables + 2 + n_outs]
    pos = n_tables + 2 + n_outs
    colbuf, wbuf = refs[pos], refs[pos + 1]
    pos += 2
    rowsb = (refs[pos:pos + n_tables], refs[pos + n_tables:pos + 2 * n_tables])
    pos += 2 * n_tables
    outb = (refs[pos:pos + n_outs], refs[pos + n_outs:pos + 2 * n_outs])
    pos += 2 * n_outs
    gsems = refs[pos:pos + 2]
    wsems = (refs[pos + 2:pos + 2 + n_outs],
             refs[pos + 2 + n_outs:pos + 2 + 2 * n_outs])
    msem = refs[pos + 2 + 2 * n_outs]
    base = _wid() * NV_PER_W

    def issue_gathers(i, p):
      return [pltpu.async_copy(
          tabs[t].at[colbuf.at[pl.ds(i * ipi, ipi)]], rowsb[p][t],
          gsems[p]) for t in range(n_tables)]

    def sup_body(s, carry):
      r0 = base + s * (supc * RPI)
      mh = [pltpu.async_copy(mcols_hbm.at[pl.ds(off + r0 * nnz, mlen)],
                             colbuf, msem),
            pltpu.async_copy(mvals_hbm.at[pl.ds(off + r0 * nnz, mlen)],
                             wbuf.at[pl.ds(0, mlen)], msem)]
      for h in mh:
        h.wait()
      gh = {0: issue_gathers(0, 0)}
      wh = {}
      for i in range(supc):
        p = i % 2
        if i + 1 < supc:
          gh[i + 1] = issue_gathers(i + 1, (i + 1) % 2)
        for h in gh.pop(i):
          h.wait()
        if i - 2 in wh:
          for h in wh.pop(i - 2):
            h.wait()

        def row_body(r, c2):
          wrow = _wvec(wbuf, i * ipi + r * nnz)
          wv = [_bcast(wrow[j]) for j in range(nnz)]
          for g in range(NGRP):
            accs = [None] * n_outs
            for j in range(nnz):
              for t in range(n_outs):
                rv = rowsb[p][min(t, n_tables - 1)][r * nnz + j,
                                                    pl.ds(g * LANES, LANES)]
                term = wv[j] * rv
                accs[t] = term if accs[t] is None else accs[t] + term
            for t in range(n_outs):
              outb[p][t][r, pl.ds(g * LANES, LANES)] = accs[t]
          return c2

        lax.fori_loop(0, RPI, row_body, 0)
        ri = r0 + i * RPI
        wh[i] = [pltpu.async_copy(outb[p][t], outs_hbm[t].at[pl.ds(ri, RPI)],
                                  wsems[p][t]) for t in range(n_outs)]
      for kk in sorted(wh):
        for h in wh.pop(kk):
          h.wait()
      return carry

    lax.fori_loop(0, n_sup, sup_body, 0)

  return comb_k


_GRAD_K = _build_grad()
_LAP_K = _build_combine(7, off=OFF_L, n_tables=1, n_outs=1)
_F2V_K = _build_combine(6, off=OFF_F, n_tables=2, n_outs=2)

_NB = 512
_NBLK = NV_PAD // _NB


def _xt_body(in_ref, o_ref):
  i = pl.program_id(0)
  t = in_ref[...].T
  rowv = lax.broadcasted_iota(jnp.int32, (_NB, C), 0) + i * _NB
  o_ref[...] = jnp.where(rowv < NV_PREV, t,
                         jnp.where(rowv < NV, 1.0, 0.0))


def _build_xt(input2d):
  n_in_blk = -(-NV_PREV // _NB) - 1
  return pl.pallas_call(
      _xt_body,
      grid=(_NBLK,),
      in_specs=[pl.BlockSpec((C, _NB),
                             lambda i: (0, jnp.minimum(i, n_in_blk)))],
      out_specs=pl.BlockSpec((_NB, C), lambda i: (i, 0)),
      out_shape=jax.ShapeDtypeStruct((NV_PAD, C), jnp.float32),
  )(input2d)


def _tc_body(x_ref, l_ref, e_ref, n_ref, c_ref, o_ref):
  feats = (x_ref, l_ref, e_ref, n_ref)
  acc = None
  for j in range(4):
    t = lax.dot_general(c_ref[j], feats[j][...], (((0,), (1,)), ((), ())),
                        preferred_element_type=jnp.float32)
    acc = t if acc is None else acc + t
  o_ref[...] = acc[None]


def _tc_matmul(x_t, lap, gv_ew, gv_ns, cj):
  feat_spec = pl.BlockSpec((_NB, 128), lambda b, i: (i, b))
  return pl.pallas_call(
      _tc_body,
      grid=(BS, _NBLK),
      in_specs=[feat_spec, feat_spec, feat_spec, feat_spec,
                pl.BlockSpec((4, 128, OUT_CH), lambda b, i: (0, 0, 0))],
      out_specs=pl.BlockSpec((1, OUT_CH, _NB), lambda b, i: (b, 0, i)),
      out_shape=jax.ShapeDtypeStruct((BS, OUT_CH, NV), jnp.float32),
  )(x_t, lap, gv_ew, gv_ns, cj)


def _pack_meta(L_cols, L_vals, F_cols, F_vals):
  zli = jnp.zeros((LSEC - NV * 7,), jnp.int32)
  zfi = jnp.zeros((FSEC - NV * 6,), jnp.int32)
  zlf = jnp.zeros((LSEC - NV * 7,), jnp.float32)
  zff = jnp.zeros((FSEC - NV * 6,), jnp.float32)
  return (jnp.concatenate([L_cols, zli, F_cols, zfi]),
          jnp.concatenate([L_vals, zlf, F_vals, zff]))


def _pack_gw(G_vals, EW, NS):
  return jnp.concatenate([G_vals, EW.reshape(-1), NS.reshape(-1)])


def kernel(input, coeffs, G_rows, G_cols, G_vals, L_rows, L_cols, L_vals,
           F_rows, F_cols, F_vals, NS, EW):
  bs, ch, _ = input.shape
  x_t = _build_xt(input.reshape(bs * ch, NV_PREV))
  mcols, mvals = _pack_meta(L_cols, L_vals, F_cols, F_vals)
  gw = _pack_gw(G_vals, EW, NS)

  gf_ew, gf_ns = _GRAD_K(x_t, G_cols, gw)
  lap = _LAP_K(x_t, mcols, mvals)
  gv_ew, gv_ns = _F2V_K(gf_ew, gf_ns, mcols, mvals)

  cj = coeffs.reshape(ch, 4, OUT_CH).transpose(1, 0, 2)
  return _tc_matmul(x_t, lap, gv_ew, gv_ns, cj)


# speedup vs baseline: 88.4487x; 1.0314x over previous
import functools

import jax
import jax.numpy as jnp
from jax import lax
from jax.experimental import pallas as pl
from jax.experimental.pallas import tpu as pltpu
from jax.experimental.pallas import tpu_sc as plsc

NV = 40962
NV_PREV = 10242
NF = 81920
C = 256
OUT_CH = 128
BS = 2
LANES = 16
NGRP = C // LANES

NC, NSUB = 2, 16
NW = NC * NSUB

RPI = 16
NV_PAD = 41472
NF_PER_W = NF // NW
NV_PER_W = NV_PAD // NW

GSEC = 3 * NF
LSEC = NV_PAD * 7
FSEC = NV_PAD * 6
OFF_L, OFF_F = 0, LSEC

_MESH = plsc.VectorSubcoreMesh(
    core_axis_name="c", subcore_axis_name="s", num_cores=NC, num_subcores=NSUB)


def _wid():
  return lax.axis_index("s") * NC + lax.axis_index("c")


def _bcast(x):
  return jnp.broadcast_to(x, (LANES,))


def _wvec(ref, off):
  return ref[pl.ds(off, LANES)]


def _build_grad(supc=8):
  n_iters = NF_PER_W // RPI
  n_sup = n_iters // supc
  sec = supc * RPI * 3
  esec = supc * RPI

  @functools.partial(
      pl.kernel, mesh=_MESH,
      out_type=(jax.ShapeDtypeStruct((NF, C), jnp.float32),
                jax.ShapeDtypeStruct((NF, C), jnp.float32)),
      scratch_types=(
          [pltpu.VMEM((3 * sec,), jnp.int32)] +
          [pltpu.VMEM((3 * sec + 6 * esec + LANES,), jnp.float32)] +
          [pltpu.VMEM((RPI * 9, C), jnp.float32) for _ in range(2)] +
          [pltpu.VMEM((RPI, C), jnp.float32) for _ in range(4)] +
          [pltpu.SemaphoreType.DMA for _ in range(7)]),
  )
  def grad_k(xt_hbm, gcols_hbm, wmeta_hbm, oew_hbm, ons_hbm,
             colbuf, wbuf, rows0, rows1, oew0, oew1, ons0, ons1,
             gsem0, gsem1, wsem_ew0, wsem_ew1, wsem_ns0, wsem_ns1, msem):
    rowsb = (rows0, rows1)
    oewb = (oew0, oew1)
    onsb = (ons0, ons1)
    gsems = (gsem0, gsem1)
    wsems = ((wsem_ew0, wsem_ns0), (wsem_ew1, wsem_ns1))
    base = _wid() * NF_PER_W

    def issue_gathers(i, p):
      return [pltpu.async_copy(
          xt_hbm.at[colbuf.at[pl.ds(d * sec + i * 48, 48)]],
          rowsb[p].at[pl.ds(d * 48, 48)], gsems[p]) for d in range(3)]

    def sup_body(s, carry):
      f0 = base + s * (supc * RPI)
      mh = [pltpu.async_copy(gcols_hbm.at[pl.ds(d * GSEC + f0 * 3, sec)],
                             colbuf.at[pl.ds(d * sec, sec)], msem)
            for d in range(3)]
      mh += [pltpu.async_copy(wmeta_hbm.at[pl.ds(d * GSEC + f0 * 3, sec)],
                              wbuf.at[pl.ds(d * sec, sec)], msem)
             for d in range(3)]
      mh += [pltpu.async_copy(
          wmeta_hbm.at[pl.ds(3 * GSEC + k * NF + f0, esec)],
          wbuf.at[pl.ds(3 * sec + k * esec, esec)], msem) for k in range(6)]
      for h in mh:
        h.wait()
      gh = {0: issue_gathers(0, 0)}
      wh = {}
      for i in range(supc):
        p = i % 2
        if i + 1 < supc:
          gh[i + 1] = issue_gathers(i + 1, (i + 1) % 2)
        for h in gh.pop(i):
          h.wait()
        if i - 2 in wh:
          for h in wh.pop(i - 2):
            h.wait()

        def row_body(r, c2):
          off = i * (RPI * 3) + r * 3
          offe = i * RPI + r
          gvv = [_wvec(wbuf, d * sec + off) for d in range(3)]
          eww = [_wvec(wbuf, 3 * sec + d * esec + offe) for d in range(3)]
          nsw = [_wvec(wbuf, 3 * sec + (3 + d) * esec + offe) for d in range(3)]
          wew = [_bcast(gvv[d][k] * eww[d][0])
                 for d in range(3) for k in range(3)]
          wns = [_bcast(gvv[d][k] * nsw[d][0])
                 for d in range(3) for k in range(3)]
          for g in range(NGRP):
            acc_ew = None
            acc_ns = None
            for d in range(3):
              for k in range(3):
                rv = rowsb[p][d * 48 + r * 3 + k, pl.ds(g * LANES, LANES)]
                tew = wew[d * 3 + k] * rv
                tns = wns[d * 3 + k] * rv
                acc_ew = tew if acc_ew is None else acc_ew + tew
                acc_ns = tns if acc_ns is None else acc_ns + tns
            oewb[p][r, pl.ds(g * LANES, LANES)] = acc_ew
            onsb[p][r, pl.ds(g * LANES, LANES)] = acc_ns
          return c2

        lax.fori_loop(0, RPI, row_body, 0)
        ri = f0 + i * RPI
        wh[i] = [
            pltpu.async_copy(oewb[p], oew_hbm.at[pl.ds(ri, RPI)], wsems[p][0]),
            pltpu.async_copy(onsb[p], ons_hbm.at[pl.ds(ri, RPI)], wsems[p][1]),
        ]
      for kk in sorted(wh):
        for h in wh.pop(kk):
          h.wait()
      return carry

    lax.fori_loop(0, n_sup, sup_body, 0)

  return grad_k


def _build_combine(nnz, off, n_tables, n_outs, supc=9):
  n_iters = NV_PER_W // RPI
  n_sup = n_iters // supc
  ipi = RPI * nnz
  mlen = supc * ipi
  assert ipi <= 128 and ipi % 8 == 0

  scratch = (
      [pltpu.VMEM((mlen,), jnp.int32)] +
      [pltpu.VMEM((mlen + LANES,), jnp.float32)] +
      [pltpu.VMEM((ipi, C), jnp.float32) for _ in range(2 * n_tables)] +
      [pltpu.VMEM((RPI, C), jnp.float32) for _ in range(2 * n_outs)] +
      [pltpu.SemaphoreType.DMA for _ in range(3 + 2 * n_outs)])
  out_types = tuple(jax.ShapeDtypeStruct((NV_PAD, C), jnp.float32)
                    for _ in range(n_outs))

  @functools.partial(pl.kernel, mesh=_MESH,
                     out_type=out_types if n_outs > 1 else out_types[0],
                     scratch_types=scratch)
  def comb_k(*refs):
    tabs = refs[:n_tables]
    mcols_hbm, mvals_hbm = refs[n_tables:n_tables + 2]
    outs_hbm = refs[n_tables + 2:n_tables + 2 + n_outs]
    pos = n_tables + 2 + n_outs
    colbuf, wbuf = refs[pos], refs[pos + 1]
    pos += 2
    rowsb = (refs[pos:pos + n_tables], refs[pos + n_tables:pos + 2 * n_tables])
    pos += 2 * n_tables
    outb = (refs[pos:pos + n_outs], refs[pos + n_outs:pos + 2 * n_outs])
    pos += 2 * n_outs
    gsems = refs[pos:pos + 2]
    wsems = (refs[pos + 2:pos + 2 + n_outs],
             refs[pos + 2 + n_outs:pos + 2 + 2 * n_outs])
    msem = refs[pos + 2 + 2 * n_outs]
    base = _wid() * NV_PER_W

    def issue_gathers(i, p):
      return [pltpu.async_copy(
          tabs[t].at[colbuf.at[pl.ds(i * ipi, ipi)]], rowsb[p][t],
          gsems[p]) for t in range(n_tables)]

    def sup_body(s, carry):
      r0 = base + s * (supc * RPI)
      mh = [pltpu.async_copy(mcols_hbm.at[pl.ds(off + r0 * nnz, mlen)],
                             colbuf, msem),
            pltpu.async_copy(mvals_hbm.at[pl.ds(off + r0 * nnz, mlen)],
                             wbuf.at[pl.ds(0, mlen)], msem)]
      for h in mh:
        h.wait()
      gh = {0: issue_gathers(0, 0)}
      wh = {}
      for i in range(supc):
        p = i % 2
        if i + 1 < supc:
          gh[i + 1] = issue_gathers(i + 1, (i + 1) % 2)
        for h in gh.pop(i):
          h.wait()
        if i - 2 in wh:
          for h in wh.pop(i - 2):
            h.wait()

        def row_body(r, c2):
          wrow = _wvec(wbuf, i * ipi + r * nnz)
          wv = [_bcast(wrow[j]) for j in range(nnz)]
          for g in range(NGRP):
            accs = [None] * n_outs
            for j in range(nnz):
              for t in range(n_outs):
                rv = rowsb[p][min(t, n_tables - 1)][r * nnz + j,
                                                    pl.ds(g * LANES, LANES)]
                term = wv[j] * rv
                accs[t] = term if accs[t] is None else accs[t] + term
            for t in range(n_outs):
              outb[p][t][r, pl.ds(g * LANES, LANES)] = accs[t]
          return c2

        lax.fori_loop(0, RPI, row_body, 0)
        ri = r0 + i * RPI
        wh[i] = [pltpu.async_copy(outb[p][t], outs_hbm[t].at[pl.ds(ri, RPI)],
                                  wsems[p][t]) for t in range(n_outs)]
      for kk in sorted(wh):
        for h in wh.pop(kk):
          h.wait()
      return carry

    lax.fori_loop(0, n_sup, sup_body, 0)

  return comb_k


_GRAD_K = _build_grad()
_LAP_K = _build_combine(7, off=OFF_L, n_tables=1, n_outs=1)
_F2V_K = _build_combine(6, off=OFF_F, n_tables=2, n_outs=2)

_NB = 512
_NBLK = NV_PAD // _NB


def _xt_body(in_ref, o_ref):
  i = pl.program_id(0)
  t = in_ref[...].T
  rowv = lax.broadcasted_iota(jnp.int32, (_NB, C), 0) + i * _NB
  o_ref[...] = jnp.where(rowv < NV_PREV, t,
                         jnp.where(rowv < NV, 1.0, 0.0))


def _build_xt(input2d):
  n_in_blk = -(-NV_PREV // _NB) - 1
  return pl.pallas_call(
      _xt_body,
      grid=(_NBLK,),
      in_specs=[pl.BlockSpec((C, _NB),
                             lambda i: (0, jnp.minimum(i, n_in_blk)))],
      out_specs=pl.BlockSpec((_NB, C), lambda i: (i, 0)),
      out_shape=jax.ShapeDtypeStruct((NV_PAD, C), jnp.float32),
  )(input2d)


def _tc_body(x_ref, l_ref, e_ref, n_ref, c_ref, o_ref):
  feats = (x_ref, l_ref, e_ref, n_ref)
  acc = None
  for j in range(4):
    t = lax.dot_general(c_ref[j], feats[j][...], (((0,), (1,)), ((), ())),
                        preferred_element_type=jnp.float32)
    acc = t if acc is None else acc + t
  o_ref[...] = acc[None]


def _tc_matmul(x_t, lap, gv_ew, gv_ns, cj):
  feat_spec = pl.BlockSpec((_NB, 128), lambda b, i: (i, b))
  return pl.pallas_call(
      _tc_body,
      grid=(BS, _NBLK),
      in_specs=[feat_spec, feat_spec, feat_spec, feat_spec,
                pl.BlockSpec((4, 128, OUT_CH), lambda b, i: (0, 0, 0))],
      out_specs=pl.BlockSpec((1, OUT_CH, _NB), lambda b, i: (b, 0, i)),
      out_shape=jax.ShapeDtypeStruct((BS, OUT_CH, NV), jnp.float32),
  )(x_t, lap, gv_ew, gv_ns, cj)


def _pack_meta(L_cols, L_vals, F_cols, F_vals):
  zli = jnp.zeros((LSEC - NV * 7,), jnp.int32)
  zfi = jnp.zeros((FSEC - NV * 6,), jnp.int32)
  zlf = jnp.zeros((LSEC - NV * 7,), jnp.float32)
  zff = jnp.zeros((FSEC - NV * 6,), jnp.float32)
  return (jnp.concatenate([L_cols, zli, F_cols, zfi]),
          jnp.concatenate([L_vals, zlf, F_vals, zff]))


def _pack_gw(G_vals, EW, NS):
  return jnp.concatenate([G_vals, EW.T.reshape(-1), NS.T.reshape(-1)])


def kernel(input, coeffs, G_rows, G_cols, G_vals, L_rows, L_cols, L_vals,
           F_rows, F_cols, F_vals, NS, EW):
  bs, ch, _ = input.shape
  x_t = _build_xt(input.reshape(bs * ch, NV_PREV))
  mcols, mvals = _pack_meta(L_cols, L_vals, F_cols, F_vals)
  gw = _pack_gw(G_vals, EW, NS)

  gf_ew, gf_ns = _GRAD_K(x_t, G_cols, gw)
  lap = _LAP_K(x_t, mcols, mvals)
  gv_ew, gv_ns = _F2V_K(gf_ew, gf_ns, mcols, mvals)

  cj = coeffs.reshape(ch, 4, OUT_CH).transpose(1, 0, 2)
  return _tc_matmul(x_t, lap, gv_ew, gv_ns, cj)


# speedup vs baseline: 90.6977x; 1.0254x over previous
import functools

import jax
import jax.numpy as jnp
from jax import lax
from jax.experimental import pallas as pl
from jax.experimental.pallas import tpu as pltpu
from jax.experimental.pallas import tpu_sc as plsc

NV = 40962
NV_PREV = 10242
NF = 81920
C = 256
OUT_CH = 128
BS = 2
LANES = 16
NGRP = C // LANES

NC, NSUB = 2, 16
NW = NC * NSUB

RPI = 16
NV_PAD = 41472
NF_PER_W = NF // NW
NV_PER_W = NV_PAD // NW

GSEC = 3 * NF
LSEC = NV_PAD * 7
FSEC = NV_PAD * 6
OFF_L, OFF_F = 0, LSEC

_MESH = plsc.VectorSubcoreMesh(
    core_axis_name="c", subcore_axis_name="s", num_cores=NC, num_subcores=NSUB)


def _wid():
  return lax.axis_index("s") * NC + lax.axis_index("c")


def _bcast(x):
  return jnp.broadcast_to(x, (LANES,))


def _wvec(ref, off):
  return ref[pl.ds(off, LANES)]


def _build_grad(supc=8):
  n_iters = NF_PER_W // RPI
  n_sup = n_iters // supc
  sec = supc * RPI * 3
  esec = supc * RPI

  @functools.partial(
      pl.kernel, mesh=_MESH,
      out_type=jax.ShapeDtypeStruct((NF, 2 * C), jnp.float32),
      scratch_types=(
          [pltpu.VMEM((3 * sec,), jnp.int32)] +
          [pltpu.VMEM((3 * sec + 6 * esec + LANES,), jnp.float32)] +
          [pltpu.VMEM((RPI * 9, C), jnp.float32) for _ in range(2)] +
          [pltpu.VMEM((RPI, C), jnp.float32) for _ in range(4)] +
          [pltpu.SemaphoreType.DMA for _ in range(7)]),
  )
  def grad_k(xt_hbm, gcols_hbm, wmeta_hbm, o_hbm,
             colbuf, wbuf, rows0, rows1, oew0, oew1, ons0, ons1,
             gsem0, gsem1, wsem_ew0, wsem_ew1, wsem_ns0, wsem_ns1, msem):
    rowsb = (rows0, rows1)
    oewb = (oew0, oew1)
    onsb = (ons0, ons1)
    gsems = (gsem0, gsem1)
    wsems = ((wsem_ew0, wsem_ns0), (wsem_ew1, wsem_ns1))
    base = _wid() * NF_PER_W

    def issue_gathers(i, p):
      return [pltpu.async_copy(
          xt_hbm.at[colbuf.at[pl.ds(d * sec + i * 48, 48)]],
          rowsb[p].at[pl.ds(d * 48, 48)], gsems[p]) for d in range(3)]

    def sup_body(s, carry):
      f0 = base + s * (supc * RPI)
      mh = [pltpu.async_copy(gcols_hbm.at[pl.ds(d * GSEC + f0 * 3, sec)],
                             colbuf.at[pl.ds(d * sec, sec)], msem)
            for d in range(3)]
      mh += [pltpu.async_copy(wmeta_hbm.at[pl.ds(d * GSEC + f0 * 3, sec)],
                              wbuf.at[pl.ds(d * sec, sec)], msem)
             for d in range(3)]
      mh += [pltpu.async_copy(
          wmeta_hbm.at[pl.ds(3 * GSEC + k * NF + f0, esec)],
          wbuf.at[pl.ds(3 * sec + k * esec, esec)], msem) for k in range(6)]
      for h in mh:
        h.wait()
      gh = {0: issue_gathers(0, 0)}
      wh = {}
      for i in range(supc):
        p = i % 2
        if i + 1 < supc:
          gh[i + 1] = issue_gathers(i + 1, (i + 1) % 2)
        for h in gh.pop(i):
          h.wait()
        if i - 2 in wh:
          for h in wh.pop(i - 2):
            h.wait()

        def row_body(r, c2):
          off = i * (RPI * 3) + r * 3
          offe = i * RPI + r
          gvv = [_wvec(wbuf, d * sec + off) for d in range(3)]
          eww = [_wvec(wbuf, 3 * sec + d * esec + offe) for d in range(3)]
          nsw = [_wvec(wbuf, 3 * sec + (3 + d) * esec + offe) for d in range(3)]
          wew = [_bcast(gvv[d][k] * eww[d][0])
                 for d in range(3) for k in range(3)]
          wns = [_bcast(gvv[d][k] * nsw[d][0])
                 for d in range(3) for k in range(3)]
          for g in range(NGRP):
            acc_ew = None
            acc_ns = None
            for d in range(3):
              for k in range(3):
                rv = rowsb[p][d * 48 + r * 3 + k, pl.ds(g * LANES, LANES)]
                tew = wew[d * 3 + k] * rv
                tns = wns[d * 3 + k] * rv
                acc_ew = tew if acc_ew is None else acc_ew + tew
                acc_ns = tns if acc_ns is None else acc_ns + tns
            oewb[p][r, pl.ds(g * LANES, LANES)] = acc_ew
            onsb[p][r, pl.ds(g * LANES, LANES)] = acc_ns
          return c2

        lax.fori_loop(0, RPI, row_body, 0)
        ri = f0 + i * RPI
        wh[i] = [
            pltpu.async_copy(oewb[p], o_hbm.at[pl.ds(ri, RPI), pl.ds(0, C)],
                             wsems[p][0]),
            pltpu.async_copy(onsb[p], o_hbm.at[pl.ds(ri, RPI), pl.ds(C, C)],
                             wsems[p][1]),
        ]
      for kk in sorted(wh):
        for h in wh.pop(kk):
          h.wait()
      return carry

    lax.fori_loop(0, n_sup, sup_body, 0)

  return grad_k


def _build_combine(nnz, off, n_outs, supc=9, tw=C):
  n_iters = NV_PER_W // RPI
  n_sup = n_iters // supc
  ipi = RPI * nnz
  mlen = supc * ipi
  assert ipi <= 128 and ipi % 8 == 0 and tw == n_outs * C

  scratch = (
      [pltpu.VMEM((mlen,), jnp.int32)] +
      [pltpu.VMEM((mlen + LANES,), jnp.float32)] +
      [pltpu.VMEM((ipi, tw), jnp.float32) for _ in range(2)] +
      [pltpu.VMEM((RPI, C), jnp.float32) for _ in range(2 * n_outs)] +
      [pltpu.SemaphoreType.DMA for _ in range(3 + 2 * n_outs)])
  out_types = tuple(jax.ShapeDtypeStruct((NV_PAD, C), jnp.float32)
                    for _ in range(n_outs))

  @functools.partial(pl.kernel, mesh=_MESH,
                     out_type=out_types if n_outs > 1 else out_types[0],
                     scratch_types=scratch)
  def comb_k(*refs):
    tab = refs[0]
    mcols_hbm, mvals_hbm = refs[1:3]
    outs_hbm = refs[3:3 + n_outs]
    pos = 3 + n_outs
    colbuf, wbuf = refs[pos], refs[pos + 1]
    pos += 2
    rowsb = refs[pos:pos + 2]
    pos += 2
    outb = (refs[pos:pos + n_outs], refs[pos + n_outs:pos + 2 * n_outs])
    pos += 2 * n_outs
    gsems = refs[pos:pos + 2]
    wsems = (refs[pos + 2:pos + 2 + n_outs],
             refs[pos + 2 + n_outs:pos + 2 + 2 * n_outs])
    msem = refs[pos + 2 + 2 * n_outs]
    base = _wid() * NV_PER_W

    def issue_gathers(i, p):
      return [pltpu.async_copy(
          tab.at[colbuf.at[pl.ds(i * ipi, ipi)]], rowsb[p], gsems[p])]

    def sup_body(s, carry):
      r0 = base + s * (supc * RPI)
      mh = [pltpu.async_copy(mcols_hbm.at[pl.ds(off + r0 * nnz, mlen)],
                             colbuf, msem),
            pltpu.async_copy(mvals_hbm.at[pl.ds(off + r0 * nnz, mlen)],
                             wbuf.at[pl.ds(0, mlen)], msem)]
      for h in mh:
        h.wait()
      gh = {0: issue_gathers(0, 0)}
      wh = {}
      for i in range(supc):
        p = i % 2
        if i + 1 < supc:
          gh[i + 1] = issue_gathers(i + 1, (i + 1) % 2)
        for h in gh.pop(i):
          h.wait()
        if i - 2 in wh:
          for h in wh.pop(i - 2):
            h.wait()

        def row_body(r, c2):
          wrow = _wvec(wbuf, i * ipi + r * nnz)
          wv = [_bcast(wrow[j]) for j in range(nnz)]
          for g in range(NGRP):
            accs = [None] * n_outs
            for j in range(nnz):
              for t in range(n_outs):
                rv = rowsb[p][r * nnz + j, pl.ds(t * C + g * LANES, LANES)]
                term = wv[j] * rv
                accs[t] = term if accs[t] is None else accs[t] + term
            for t in range(n_outs):
              outb[p][t][r, pl.ds(g * LANES, LANES)] = accs[t]
          return c2

        lax.fori_loop(0, RPI, row_body, 0)
        ri = r0 + i * RPI
        wh[i] = [pltpu.async_copy(outb[p][t], outs_hbm[t].at[pl.ds(ri, RPI)],
                                  wsems[p][t]) for t in range(n_outs)]
      for kk in sorted(wh):
        for h in wh.pop(kk):
          h.wait()
      return carry

    lax.fori_loop(0, n_sup, sup_body, 0)

  return comb_k


_GRAD_K = _build_grad()
_LAP_K = _build_combine(7, off=OFF_L, n_outs=1, tw=C)
_F2V_K = _build_combine(6, off=OFF_F, n_outs=2, tw=2 * C)

_NB = 512
_NBLK = NV_PAD // _NB


def _xt_body(in_ref, o_ref):
  i = pl.program_id(0)
  t = in_ref[...].T
  rowv = lax.broadcasted_iota(jnp.int32, (_NB, C), 0) + i * _NB
  o_ref[...] = jnp.where(rowv < NV_PREV, t,
                         jnp.where(rowv < NV, 1.0, 0.0))


def _build_xt(input2d):
  n_in_blk = -(-NV_PREV // _NB) - 1
  return pl.pallas_call(
      _xt_body,
      grid=(_NBLK,),
      in_specs=[pl.BlockSpec((C, _NB),
                             lambda i: (0, jnp.minimum(i, n_in_blk)))],
      out_specs=pl.BlockSpec((_NB, C), lambda i: (i, 0)),
      out_shape=jax.ShapeDtypeStruct((NV_PAD, C), jnp.float32),
  )(input2d)


def _tc_body(x_ref, l_ref, e_ref, n_ref, c_ref, o_ref):
  feats = (x_ref, l_ref, e_ref, n_ref)
  acc = None
  for j in range(4):
    t = lax.dot_general(c_ref[j], feats[j][...], (((0,), (1,)), ((), ())),
                        preferred_element_type=jnp.float32)
    acc = t if acc is None else acc + t
  o_ref[...] = acc[None]


def _tc_matmul(x_t, lap, gv_ew, gv_ns, cj):
  feat_spec = pl.BlockSpec((_NB, 128), lambda b, i: (i, b))
  return pl.pallas_call(
      _tc_body,
      grid=(BS, _NBLK),
      in_specs=[feat_spec, feat_spec, feat_spec, feat_spec,
                pl.BlockSpec((4, 128, OUT_CH), lambda b, i: (0, 0, 0))],
      out_specs=pl.BlockSpec((1, OUT_CH, _NB), lambda b, i: (b, 0, i)),
      out_shape=jax.ShapeDtypeStruct((BS, OUT_CH, NV), jnp.float32),
  )(x_t, lap, gv_ew, gv_ns, cj)


def _pack_meta(L_cols, L_vals, F_cols, F_vals):
  zli = jnp.zeros((LSEC - NV * 7,), jnp.int32)
  zfi = jnp.zeros((FSEC - NV * 6,), jnp.int32)
  zlf = jnp.zeros((LSEC - NV * 7,), jnp.float32)
  zff = jnp.zeros((FSEC - NV * 6,), jnp.float32)
  return (jnp.concatenate([L_cols, zli, F_cols, zfi]),
          jnp.concatenate([L_vals, zlf, F_vals, zff]))


def _pack_gw(G_vals, EW, NS):
  return jnp.concatenate([G_vals, EW.T.reshape(-1), NS.T.reshape(-1)])


def kernel(input, coeffs, G_rows, G_cols, G_vals, L_rows, L_cols, L_vals,
           F_rows, F_cols, F_vals, NS, EW):
  bs, ch, _ = input.shape
  x_t = _build_xt(input.reshape(bs * ch, NV_PREV))
  mcols, mvals = _pack_meta(L_cols, L_vals, F_cols, F_vals)
  gw = _pack_gw(G_vals, EW, NS)

  gf = _GRAD_K(x_t, G_cols, gw)
  lap = _LAP_K(x_t, mcols, mvals)
  gv_ew, gv_ns = _F2V_K(gf, mcols, mvals)

  cj = coeffs.reshape(ch, 4, OUT_CH).transpose(1, 0, 2)
  return _tc_matmul(x_t, lap, gv_ew, gv_ns, cj)


# speedup vs baseline: 90.9733x; 1.0030x over previous
import functools

import jax
import jax.numpy as jnp
from jax import lax
from jax.experimental import pallas as pl
from jax.experimental.pallas import tpu as pltpu
from jax.experimental.pallas import tpu_sc as plsc

NV = 40962
NV_PREV = 10242
NF = 81920
C = 256
OUT_CH = 128
BS = 2
LANES = 16
NGRP = C // LANES

NC, NSUB = 2, 16
NW = NC * NSUB

RPI = 16
NV_PAD = 41472
NF_PER_W = NF // NW
NV_PER_W = NV_PAD // NW

GSEC = 3 * NF
LSEC = NV_PAD * 7
FSEC = NV_PAD * 6
OFF_L, OFF_F = 0, LSEC

_MESH = plsc.VectorSubcoreMesh(
    core_axis_name="c", subcore_axis_name="s", num_cores=NC, num_subcores=NSUB)


def _wid():
  return lax.axis_index("s") * NC + lax.axis_index("c")


def _bcast(x):
  return jnp.broadcast_to(x, (LANES,))


def _wvec(ref, off):
  return ref[pl.ds(off, LANES)]


def _build_grad(supc=8):
  n_iters = NF_PER_W // RPI
  n_sup = n_iters // supc
  sec = supc * RPI * 3
  esec = supc * RPI

  @functools.partial(
      pl.kernel, mesh=_MESH,
      out_type=jax.ShapeDtypeStruct((NF, 2 * C), jnp.float32),
      scratch_types=(
          [pltpu.VMEM((3 * sec,), jnp.int32)] +
          [pltpu.VMEM((3 * sec + 6 * esec + LANES,), jnp.float32)] +
          [pltpu.VMEM((RPI * 9, C), jnp.float32) for _ in range(2)] +
          [pltpu.VMEM((RPI, C), jnp.float32) for _ in range(4)] +
          [pltpu.SemaphoreType.DMA for _ in range(7)]),
  )
  def grad_k(xt_hbm, gcols_hbm, wmeta_hbm, o_hbm,
             colbuf, wbuf, rows0, rows1, oew0, oew1, ons0, ons1,
             gsem0, gsem1, wsem_ew0, wsem_ew1, wsem_ns0, wsem_ns1, msem):
    rowsb = (rows0, rows1)
    oewb = (oew0, oew1)
    onsb = (ons0, ons1)
    gsems = (gsem0, gsem1)
    wsems = ((wsem_ew0, wsem_ns0), (wsem_ew1, wsem_ns1))
    base = _wid() * NF_PER_W

    def issue_gathers(i, p):
      return [pltpu.async_copy(
          xt_hbm.at[colbuf.at[pl.ds(d * sec + i * 48, 48)]],
          rowsb[p].at[pl.ds(d * 48, 48)], gsems[p]) for d in range(3)]

    def issue_meta(f0):
      for d in range(3):
        pltpu.async_copy(gcols_hbm.at[pl.ds(d * GSEC + f0 * 3, sec)],
                         colbuf.at[pl.ds(d * sec, sec)], msem)
        pltpu.async_copy(wmeta_hbm.at[pl.ds(d * GSEC + f0 * 3, sec)],
                         wbuf.at[pl.ds(d * sec, sec)], msem)
      for k in range(6):
        pltpu.async_copy(wmeta_hbm.at[pl.ds(3 * GSEC + k * NF + f0, esec)],
                         wbuf.at[pl.ds(3 * sec + k * esec, esec)], msem)

    def wait_meta(f0):
      for d in range(3):
        pltpu.make_async_copy(gcols_hbm.at[pl.ds(d * GSEC + f0 * 3, sec)],
                              colbuf.at[pl.ds(d * sec, sec)], msem).wait()
        pltpu.make_async_copy(wmeta_hbm.at[pl.ds(d * GSEC + f0 * 3, sec)],
                              wbuf.at[pl.ds(d * sec, sec)], msem).wait()
      for k in range(6):
        pltpu.make_async_copy(
            wmeta_hbm.at[pl.ds(3 * GSEC + k * NF + f0, esec)],
            wbuf.at[pl.ds(3 * sec + k * esec, esec)], msem).wait()

    def sup_body(s, carry):
      f0 = base + s * (supc * RPI)

      @pl.when(s == 0)
      def _prologue():
        issue_meta(f0)

      wait_meta(f0)
      gh = {0: issue_gathers(0, 0)}
      wh = {}
      for i in range(supc):
        p = i % 2
        if i + 1 < supc:
          gh[i + 1] = issue_gathers(i + 1, (i + 1) % 2)
        for h in gh.pop(i):
          h.wait()
        if i - 2 in wh:
          for h in wh.pop(i - 2):
            h.wait()

        def row_body(r, c2):
          off = i * (RPI * 3) + r * 3
          offe = i * RPI + r
          gvv = [_wvec(wbuf, d * sec + off) for d in range(3)]
          eww = [_wvec(wbuf, 3 * sec + d * esec + offe) for d in range(3)]
          nsw = [_wvec(wbuf, 3 * sec + (3 + d) * esec + offe) for d in range(3)]
          wew = [_bcast(gvv[d][k] * eww[d][0])
                 for d in range(3) for k in range(3)]
          wns = [_bcast(gvv[d][k] * nsw[d][0])
                 for d in range(3) for k in range(3)]
          for g in range(NGRP):
            acc_ew = None
            acc_ns = None
            for d in range(3):
              for k in range(3):
                rv = rowsb[p][d * 48 + r * 3 + k, pl.ds(g * LANES, LANES)]
                tew = wew[d * 3 + k] * rv
                tns = wns[d * 3 + k] * rv
                acc_ew = tew if acc_ew is None else acc_ew + tew
                acc_ns = tns if acc_ns is None else acc_ns + tns
            oewb[p][r, pl.ds(g * LANES, LANES)] = acc_ew
            onsb[p][r, pl.ds(g * LANES, LANES)] = acc_ns
          return c2

        lax.fori_loop(0, RPI, row_body, 0)
        ri = f0 + i * RPI
        wh[i] = [
            pltpu.async_copy(oewb[p], o_hbm.at[pl.ds(ri, RPI), pl.ds(0, C)],
                             wsems[p][0]),
            pltpu.async_copy(onsb[p], o_hbm.at[pl.ds(ri, RPI), pl.ds(C, C)],
                             wsems[p][1]),
        ]

      @pl.when(s + 1 < n_sup)
      def _prefetch():
        issue_meta(base + (s + 1) * (supc * RPI))

      for kk in sorted(wh):
        for h in wh.pop(kk):
          h.wait()
      return carry

    lax.fori_loop(0, n_sup, sup_body, 0)

  return grad_k


def _build_combine(nnz, off, n_outs, supc=9, tw=C):
  n_iters = NV_PER_W // RPI
  n_sup = n_iters // supc
  ipi = RPI * nnz
  mlen = supc * ipi
  assert ipi <= 128 and ipi % 8 == 0 and tw == n_outs * C

  scratch = (
      [pltpu.VMEM((mlen,), jnp.int32)] +
      [pltpu.VMEM((mlen + LANES,), jnp.float32)] +
      [pltpu.VMEM((ipi, tw), jnp.float32) for _ in range(2)] +
      [pltpu.VMEM((RPI, C), jnp.float32) for _ in range(2 * n_outs)] +
      [pltpu.SemaphoreType.DMA for _ in range(3 + 2 * n_outs)])
  out_types = tuple(jax.ShapeDtypeStruct((NV_PAD, C), jnp.float32)
                    for _ in range(n_outs))

  @functools.partial(pl.kernel, mesh=_MESH,
                     out_type=out_types if n_outs > 1 else out_types[0],
                     scratch_types=scratch)
  def comb_k(*refs):
    tab = refs[0]
    mcols_hbm, mvals_hbm = refs[1:3]
    outs_hbm = refs[3:3 + n_outs]
    pos = 3 + n_outs
    colbuf, wbuf = refs[pos], refs[pos + 1]
    pos += 2
    rowsb = refs[pos:pos + 2]
    pos += 2
    outb = (refs[pos:pos + n_outs], refs[pos + n_outs:pos + 2 * n_outs])
    pos += 2 * n_outs
    gsems = refs[pos:pos + 2]
    wsems = (refs[pos + 2:pos + 2 + n_outs],
             refs[pos + 2 + n_outs:pos + 2 + 2 * n_outs])
    msem = refs[pos + 2 + 2 * n_outs]
    base = _wid() * NV_PER_W

    def issue_gathers(i, p):
      return [pltpu.async_copy(
          tab.at[colbuf.at[pl.ds(i * ipi, ipi)]], rowsb[p], gsems[p])]

    def issue_meta(r0):
      pltpu.async_copy(mcols_hbm.at[pl.ds(off + r0 * nnz, mlen)], colbuf,
                       msem)
      pltpu.async_copy(mvals_hbm.at[pl.ds(off + r0 * nnz, mlen)],
                       wbuf.at[pl.ds(0, mlen)], msem)

    def wait_meta(r0):
      pltpu.make_async_copy(mcols_hbm.at[pl.ds(off + r0 * nnz, mlen)],
                            colbuf, msem).wait()
      pltpu.make_async_copy(mvals_hbm.at[pl.ds(off + r0 * nnz, mlen)],
                            wbuf.at[pl.ds(0, mlen)], msem).wait()

    def sup_body(s, carry):
      r0 = base + s * (supc * RPI)

      @pl.when(s == 0)
      def _prologue():
        issue_meta(r0)

      wait_meta(r0)
      gh = {0: issue_gathers(0, 0)}
      wh = {}
      for i in range(supc):
        p = i % 2
        if i + 1 < supc:
          gh[i + 1] = issue_gathers(i + 1, (i + 1) % 2)
        for h in gh.pop(i):
          h.wait()
        if i - 2 in wh:
          for h in wh.pop(i - 2):
            h.wait()

        def row_body(r, c2):
          wrow = _wvec(wbuf, i * ipi + r * nnz)
          wv = [_bcast(wrow[j]) for j in range(nnz)]
          for g in range(NGRP):
            accs = [None] * n_outs
            for j in range(nnz):
              for t in range(n_outs):
                rv = rowsb[p][r * nnz + j, pl.ds(t * C + g * LANES, LANES)]
                term = wv[j] * rv
                accs[t] = term if accs[t] is None else accs[t] + term
            for t in range(n_outs):
              outb[p][t][r, pl.ds(g * LANES, LANES)] = accs[t]
          return c2

        lax.fori_loop(0, RPI, row_body, 0)
        ri = r0 + i * RPI
        wh[i] = [pltpu.async_copy(outb[p][t], outs_hbm[t].at[pl.ds(ri, RPI)],
                                  wsems[p][t]) for t in range(n_outs)]

      @pl.when(s + 1 < n_sup)
      def _prefetch():
        issue_meta(base + (s + 1) * (supc * RPI))

      for kk in sorted(wh):
        for h in wh.pop(kk):
          h.wait()
      return carry

    lax.fori_loop(0, n_sup, sup_body, 0)

  return comb_k


_GRAD_K = _build_grad()
_LAP_K = _build_combine(7, off=OFF_L, n_outs=1, tw=C)
_F2V_K = _build_combine(6, off=OFF_F, n_outs=2, tw=2 * C)

_NB = 512
_NBLK = NV_PAD // _NB


def _xt_body(in_ref, o_ref):
  i = pl.program_id(0)
  t = in_ref[...].T
  rowv = lax.broadcasted_iota(jnp.int32, (_NB, C), 0) + i * _NB
  o_ref[...] = jnp.where(rowv < NV_PREV, t,
                         jnp.where(rowv < NV, 1.0, 0.0))


def _build_xt(input2d):
  n_in_blk = -(-NV_PREV // _NB) - 1
  return pl.pallas_call(
      _xt_body,
      grid=(_NBLK,),
      in_specs=[pl.BlockSpec((C, _NB),
                             lambda i: (0, jnp.minimum(i, n_in_blk)))],
      out_specs=pl.BlockSpec((_NB, C), lambda i: (i, 0)),
      out_shape=jax.ShapeDtypeStruct((NV_PAD, C), jnp.float32),
  )(input2d)


def _tc_body(x_ref, l_ref, e_ref, n_ref, c_ref, o_ref):
  feats = (x_ref, l_ref, e_ref, n_ref)
  acc = None
  for j in range(4):
    t = lax.dot_general(c_ref[j], feats[j][...], (((0,), (1,)), ((), ())),
                        preferred_element_type=jnp.float32)
    acc = t if acc is None else acc + t
  o_ref[...] = acc[None]


def _tc_matmul(x_t, lap, gv_ew, gv_ns, cj):
  feat_spec = pl.BlockSpec((_NB, 128), lambda b, i: (i, b))
  return pl.pallas_call(
      _tc_body,
      grid=(BS, _NBLK),
      in_specs=[feat_spec, feat_spec, feat_spec, feat_spec,
                pl.BlockSpec((4, 128, OUT_CH), lambda b, i: (0, 0, 0))],
      out_specs=pl.BlockSpec((1, OUT_CH, _NB), lambda b, i: (b, 0, i)),
      out_shape=jax.ShapeDtypeStruct((BS, OUT_CH, NV), jnp.float32),
  )(x_t, lap, gv_ew, gv_ns, cj)


def _pack_meta(L_cols, L_vals, F_cols, F_vals):
  zli = jnp.zeros((LSEC - NV * 7,), jnp.int32)
  zfi = jnp.zeros((FSEC - NV * 6,), jnp.int32)
  zlf = jnp.zeros((LSEC - NV * 7,), jnp.float32)
  zff = jnp.zeros((FSEC - NV * 6,), jnp.float32)
  return (jnp.concatenate([L_cols, zli, F_cols, zfi]),
          jnp.concatenate([L_vals, zlf, F_vals, zff]))


def _pack_gw(G_vals, EW, NS):
  return jnp.concatenate([G_vals, EW.T.reshape(-1), NS.T.reshape(-1)])


def kernel(input, coeffs, G_rows, G_cols, G_vals, L_rows, L_cols, L_vals,
           F_rows, F_cols, F_vals, NS, EW):
  bs, ch, _ = input.shape
  x_t = _build_xt(input.reshape(bs * ch, NV_PREV))
  mcols, mvals = _pack_meta(L_cols, L_vals, F_cols, F_vals)
  gw = _pack_gw(G_vals, EW, NS)

  gf = _GRAD_K(x_t, G_cols, gw)
  lap = _LAP_K(x_t, mcols, mvals)
  gv_ew, gv_ns = _F2V_K(gf, mcols, mvals)

  cj = coeffs.reshape(ch, 4, OUT_CH).transpose(1, 0, 2)
  return _tc_matmul(x_t, lap, gv_ew, gv_ns, cj)


# speedup vs baseline: 91.9227x; 1.0104x over previous
import functools

import jax
import jax.numpy as jnp
from jax import lax
from jax.experimental import pallas as pl
from jax.experimental.pallas import tpu as pltpu
from jax.experimental.pallas import tpu_sc as plsc

NV = 40962
NV_PREV = 10242
NF = 81920
C = 256
OUT_CH = 128
BS = 2
LANES = 16
NGRP = C // LANES

NC, NSUB = 2, 16
NW = NC * NSUB

RPI = 16
NV_PAD = 41472
NF_PER_W = NF // NW
NV_PER_W = NV_PAD // NW

GSEC = 3 * NF
LSEC = NV_PAD * 7
FSEC = NV_PAD * 6
OFF_L, OFF_F = 0, LSEC

_MESH = plsc.VectorSubcoreMesh(
    core_axis_name="c", subcore_axis_name="s", num_cores=NC, num_subcores=NSUB)


def _wid():
  return lax.axis_index("s") * NC + lax.axis_index("c")


def _bcast(x):
  return jnp.broadcast_to(x, (LANES,))


def _wvec(ref, off):
  return ref[pl.ds(off, LANES)]


def _build_grad(supc=8):
  n_iters = NF_PER_W // RPI
  n_sup = n_iters // supc
  sec = supc * RPI * 3
  esec = supc * RPI

  @functools.partial(
      pl.kernel, mesh=_MESH,
      out_type=jax.ShapeDtypeStruct((NF, 2 * C), jnp.float32),
      scratch_types=(
          [pltpu.VMEM((3 * sec,), jnp.int32)] +
          [pltpu.VMEM((3 * sec + 6 * esec + LANES,), jnp.float32)] +
          [pltpu.VMEM((RPI * 9, C), jnp.float32) for _ in range(2)] +
          [pltpu.VMEM((RPI, C), jnp.float32) for _ in range(4)] +
          [pltpu.SemaphoreType.DMA for _ in range(7)]),
  )
  def grad_k(xt_hbm, gcols_hbm, wmeta_hbm, o_hbm,
             colbuf, wbuf, rows0, rows1, oew0, oew1, ons0, ons1,
             gsem0, gsem1, wsem_ew0, wsem_ew1, wsem_ns0, wsem_ns1, msem):
    rowsb = (rows0, rows1)
    oewb = (oew0, oew1)
    onsb = (ons0, ons1)
    gsems = (gsem0, gsem1)
    wsems = ((wsem_ew0, wsem_ns0), (wsem_ew1, wsem_ns1))
    base = _wid() * NF_PER_W

    def issue_gathers(i, p):
      return [pltpu.async_copy(
          xt_hbm.at[colbuf.at[pl.ds(d * sec + i * 48, 48)]],
          rowsb[p].at[pl.ds(d * 48, 48)], gsems[p]) for d in range(3)]

    def issue_meta(f0):
      for d in range(3):
        pltpu.async_copy(gcols_hbm.at[pl.ds(d * GSEC + f0 * 3, sec)],
                         colbuf.at[pl.ds(d * sec, sec)], msem)
        pltpu.async_copy(wmeta_hbm.at[pl.ds(d * GSEC + f0 * 3, sec)],
                         wbuf.at[pl.ds(d * sec, sec)], msem)
      for k in range(6):
        pltpu.async_copy(wmeta_hbm.at[pl.ds(3 * GSEC + k * NF + f0, esec)],
                         wbuf.at[pl.ds(3 * sec + k * esec, esec)], msem)

    def wait_meta(f0):
      for d in range(3):
        pltpu.make_async_copy(gcols_hbm.at[pl.ds(d * GSEC + f0 * 3, sec)],
                              colbuf.at[pl.ds(d * sec, sec)], msem).wait()
        pltpu.make_async_copy(wmeta_hbm.at[pl.ds(d * GSEC + f0 * 3, sec)],
                              wbuf.at[pl.ds(d * sec, sec)], msem).wait()
      for k in range(6):
        pltpu.make_async_copy(
            wmeta_hbm.at[pl.ds(3 * GSEC + k * NF + f0, esec)],
            wbuf.at[pl.ds(3 * sec + k * esec, esec)], msem).wait()

    def sup_body(s, carry):
      f0 = base + s * (supc * RPI)

      @pl.when(s == 0)
      def _prologue():
        issue_meta(f0)

      wait_meta(f0)
      gh = {0: issue_gathers(0, 0)}
      wh = {}
      for i in range(supc):
        p = i % 2
        if i + 1 < supc:
          gh[i + 1] = issue_gathers(i + 1, (i + 1) % 2)
        for h in gh.pop(i):
          h.wait()
        if i - 2 in wh:
          for h in wh.pop(i - 2):
            h.wait()

        def row_body(r, c2):
          off = i * (RPI * 3) + r * 3
          offe = i * RPI + r
          gvv = [_wvec(wbuf, d * sec + off) for d in range(3)]
          eww = [_wvec(wbuf, 3 * sec + d * esec + offe) for d in range(3)]
          nsw = [_wvec(wbuf, 3 * sec + (3 + d) * esec + offe) for d in range(3)]
          wew = [_bcast(gvv[d][k] * eww[d][0])
                 for d in range(3) for k in range(3)]
          wns = [_bcast(gvv[d][k] * nsw[d][0])
                 for d in range(3) for k in range(3)]
          for g in range(NGRP):
            acc_ew = None
            acc_ns = None
            for d in range(3):
              for k in range(3):
                rv = rowsb[p][d * 48 + r * 3 + k, pl.ds(g * LANES, LANES)]
                tew = wew[d * 3 + k] * rv
                tns = wns[d * 3 + k] * rv
                acc_ew = tew if acc_ew is None else acc_ew + tew
                acc_ns = tns if acc_ns is None else acc_ns + tns
            oewb[p][r, pl.ds(g * LANES, LANES)] = acc_ew
            onsb[p][r, pl.ds(g * LANES, LANES)] = acc_ns
          return c2

        lax.fori_loop(0, RPI, row_body, 0)
        ri = f0 + i * RPI
        wh[i] = [
            pltpu.async_copy(oewb[p], o_hbm.at[pl.ds(ri, RPI), pl.ds(0, C)],
                             wsems[p][0]),
            pltpu.async_copy(onsb[p], o_hbm.at[pl.ds(ri, RPI), pl.ds(C, C)],
                             wsems[p][1]),
        ]

      @pl.when(s + 1 < n_sup)
      def _prefetch():
        issue_meta(base + (s + 1) * (supc * RPI))

      for kk in sorted(wh):
        for h in wh.pop(kk):
          h.wait()
      return carry

    lax.fori_loop(0, n_sup, sup_body, 0)

  return grad_k


def _build_combine(nnz, off, n_outs, supc=9, tw=C):
  n_iters = NV_PER_W // RPI
  n_sup = n_iters // supc
  ipi = RPI * nnz
  mlen = supc * ipi
  assert ipi <= 128 and ipi % 8 == 0 and tw == n_outs * C

  scratch = (
      [pltpu.VMEM((mlen,), jnp.int32)] +
      [pltpu.VMEM((mlen + LANES,), jnp.float32)] +
      [pltpu.VMEM((ipi, tw), jnp.float32) for _ in range(2)] +
      [pltpu.VMEM((RPI, C), jnp.float32) for _ in range(2 * n_outs)] +
      [pltpu.SemaphoreType.DMA for _ in range(3 + 2 * n_outs)])
  out_types = tuple(jax.ShapeDtypeStruct((NV_PAD, C), jnp.float32)
                    for _ in range(n_outs))

  @functools.partial(pl.kernel, mesh=_MESH,
                     out_type=out_types if n_outs > 1 else out_types[0],
                     scratch_types=scratch)
  def comb_k(*refs):
    tab = refs[0]
    mcols_hbm, mvals_hbm = refs[1:3]
    outs_hbm = refs[3:3 + n_outs]
    pos = 3 + n_outs
    colbuf, wbuf = refs[pos], refs[pos + 1]
    pos += 2
    rowsb = refs[pos:pos + 2]
    pos += 2
    outb = (refs[pos:pos + n_outs], refs[pos + n_outs:pos + 2 * n_outs])
    pos += 2 * n_outs
    gsems = refs[pos:pos + 2]
    wsems = (refs[pos + 2:pos + 2 + n_outs],
             refs[pos + 2 + n_outs:pos + 2 + 2 * n_outs])
    msem = refs[pos + 2 + 2 * n_outs]
    base = _wid() * NV_PER_W

    def issue_gathers(i, p):
      return [pltpu.async_copy(
          tab.at[colbuf.at[pl.ds(i * ipi, ipi)]], rowsb[p], gsems[p])]

    def issue_meta(r0):
      pltpu.async_copy(mcols_hbm.at[pl.ds(off + r0 * nnz, mlen)], colbuf,
                       msem)
      pltpu.async_copy(mvals_hbm.at[pl.ds(off + r0 * nnz, mlen)],
                       wbuf.at[pl.ds(0, mlen)], msem)

    def wait_meta(r0):
      pltpu.make_async_copy(mcols_hbm.at[pl.ds(off + r0 * nnz, mlen)],
                            colbuf, msem).wait()
      pltpu.make_async_copy(mvals_hbm.at[pl.ds(off + r0 * nnz, mlen)],
                            wbuf.at[pl.ds(0, mlen)], msem).wait()

    def sup_body(s, carry):
      r0 = base + s * (supc * RPI)

      @pl.when(s == 0)
      def _prologue():
        issue_meta(r0)

      wait_meta(r0)
      gh = {0: issue_gathers(0, 0)}
      wh = {}
      for i in range(supc):
        p = i % 2
        if i + 1 < supc:
          gh[i + 1] = issue_gathers(i + 1, (i + 1) % 2)
        for h in gh.pop(i):
          h.wait()
        if i - 2 in wh:
          for h in wh.pop(i - 2):
            h.wait()

        def row_body(r, c2):
          wrow = _wvec(wbuf, i * ipi + r * nnz)
          wv = [_bcast(wrow[j]) for j in range(nnz)]
          for g in range(NGRP):
            accs = [None] * n_outs
            for j in range(nnz):
              for t in range(n_outs):
                rv = rowsb[p][r * nnz + j, pl.ds(t * C + g * LANES, LANES)]
                term = wv[j] * rv
                accs[t] = term if accs[t] is None else accs[t] + term
            for t in range(n_outs):
              outb[p][t][r, pl.ds(g * LANES, LANES)] = accs[t]
          return c2

        lax.fori_loop(0, RPI, row_body, 0)
        ri = r0 + i * RPI
        wh[i] = [pltpu.async_copy(outb[p][t], outs_hbm[t].at[pl.ds(ri, RPI)],
                                  wsems[p][t]) for t in range(n_outs)]

      @pl.when(s + 1 < n_sup)
      def _prefetch():
        issue_meta(base + (s + 1) * (supc * RPI))

      for kk in sorted(wh):
        for h in wh.pop(kk):
          h.wait()
      return carry

    lax.fori_loop(0, n_sup, sup_body, 0)

  return comb_k


_GRAD_K = _build_grad()
_LAP_K = _build_combine(7, off=OFF_L, n_outs=1, tw=C)
_F2V_K = _build_combine(6, off=OFF_F, n_outs=2, tw=2 * C)

_NB = 512
_NBLK = NV_PAD // _NB


def _xt_body(in_ref, o_ref):
  i = pl.program_id(0)
  t = in_ref[...].T
  rowv = lax.broadcasted_iota(jnp.int32, (_NB, C), 0) + i * _NB
  o_ref[...] = jnp.where(rowv < NV_PREV, t,
                         jnp.where(rowv < NV, 1.0, 0.0))


def _build_xt(input2d):
  n_in_blk = -(-NV_PREV // _NB) - 1
  return pl.pallas_call(
      _xt_body,
      grid=(_NBLK,),
      in_specs=[pl.BlockSpec((C, _NB),
                             lambda i: (0, jnp.minimum(i, n_in_blk)))],
      out_specs=pl.BlockSpec((_NB, C), lambda i: (i, 0)),
      out_shape=jax.ShapeDtypeStruct((NV_PAD, C), jnp.float32),
  )(input2d)


def _tc_body(x_ref, l_ref, e_ref, n_ref, c_ref, o_ref):
  feats = (x_ref, l_ref, e_ref, n_ref)
  acc = None
  for j in range(4):
    t = lax.dot_general(feats[j][...], c_ref[j], (((1,), (0,)), ((), ())),
                        preferred_element_type=jnp.float32)
    acc = t if acc is None else acc + t
  o_ref[...] = acc


def _tc_matmul(x_t, lap, gv_ew, gv_ns, cj):
  feat_spec = pl.BlockSpec((_NB, 128), lambda b, i: (i, b))
  return pl.pallas_call(
      _tc_body,
      grid=(BS, _NBLK),
      in_specs=[feat_spec, feat_spec, feat_spec, feat_spec,
                pl.BlockSpec((4, 128, OUT_CH), lambda b, i: (0, 0, 0))],
      out_specs=pl.BlockSpec((_NB, OUT_CH), lambda b, i: (i, b)),
      out_shape=jax.ShapeDtypeStruct((NV, BS * OUT_CH), jnp.float32),
  )(x_t, lap, gv_ew, gv_ns, cj)


def _pack_meta(L_cols, L_vals, F_cols, F_vals):
  zli = jnp.zeros((LSEC - NV * 7,), jnp.int32)
  zfi = jnp.zeros((FSEC - NV * 6,), jnp.int32)
  zlf = jnp.zeros((LSEC - NV * 7,), jnp.float32)
  zff = jnp.zeros((FSEC - NV * 6,), jnp.float32)
  return (jnp.concatenate([L_cols, zli, F_cols, zfi]),
          jnp.concatenate([L_vals, zlf, F_vals, zff]))


def _pack_gw(G_vals, EW, NS):
  return jnp.concatenate([G_vals, EW.T.reshape(-1), NS.T.reshape(-1)])


def kernel(input, coeffs, G_rows, G_cols, G_vals, L_rows, L_cols, L_vals,
           F_rows, F_cols, F_vals, NS, EW):
  bs, ch, _ = input.shape
  x_t = _build_xt(input.reshape(bs * ch, NV_PREV))
  mcols, mvals = _pack_meta(L_cols, L_vals, F_cols, F_vals)
  gw = _pack_gw(G_vals, EW, NS)

  gf = _GRAD_K(x_t, G_cols, gw)
  lap = _LAP_K(x_t, mcols, mvals)
  gv_ew, gv_ns = _F2V_K(gf, mcols, mvals)

  cj = coeffs.reshape(ch, 4, OUT_CH).transpose(1, 0, 2)
  out_v = _tc_matmul(x_t, lap, gv_ew, gv_ns, cj)
  return out_v.reshape(NV, bs, OUT_CH).transpose(1, 2, 0)
